# Initial kernel scaffold; baseline (speedup 1.0000x reference)
import jax
import jax.numpy as jnp
from jax.experimental import pallas as pl


def kernel(feature, edge_index, mirna_sim, disease_sim, W1, a1_src, a1_dst, b1, W2, a2_src, a2_dst, b2, W3, a3_src, a3_dst, b3, att_m, att_d, alpha1, alpha2):
    raise NotImplementedError("write your pallas kernel here")


# speedup vs baseline: 17.4605x; 17.4605x over previous
import dataclasses
import functools

import jax
import jax.numpy as jnp
from jax import lax
from jax.experimental import pallas as pl
from jax.experimental.pallas import tpu as pltpu
from jax.experimental.pallas import tpu_sc as plsc

M_SIZE = 2048
D_SIZE = 2048
N = M_SIZE + D_SIZE
E0 = 131072
E = E0 + N
F1, F2, F3 = 128, 64, 32
GAMMAS = (0.03125, 0.03125, 0.03125)
NEG_SLOPE = 0.2

NC, NS, LN = 2, 16, 16
NW = NC * NS
EPW = E // NW
CHUNK = 128
NCHUNK = EPW // CHUNK

XP = 16

f32 = jnp.float32
HIGH = lax.Precision.HIGHEST


def _linear_body(nk, F, x_ref, w_ref, a_ref, haug_ref, s2_ref):
    k = pl.program_id(1)

    @pl.when(k == 0)
    def _():
        haug_ref[...] = jnp.zeros_like(haug_ref)

    h_part = lax.dot_general(
        x_ref[...], w_ref[...], (((1,), (0,)), ((), ())),
        precision=HIGH, preferred_element_type=f32)
    haug_ref[:, :F] += h_part

    @pl.when(k == nk - 1)
    def _():
        ones_col = jnp.where(
            lax.broadcasted_iota(jnp.int32, (haug_ref.shape[0], XP), 1) == 0,
            1.0, 0.0)
        haug_ref[:, F:] = ones_col
        h_full = haug_ref[:, :F]
        s2_ref[...] = lax.dot_general(
            a_ref[...], h_full, (((0,), (1,)), ((), ())),
            precision=HIGH, preferred_element_type=f32)


def tc_linear(x, w, a_src, a_dst):
    K, F = w.shape
    bm = 256
    bk = min(K, 512)
    nk = K // bk
    amat = jnp.concatenate(
        [a_src[:, None], a_dst[:, None], jnp.zeros((F, 14), f32)], axis=1)
    return pl.pallas_call(
        functools.partial(_linear_body, nk, F),
        grid=(N // bm, nk),
        in_specs=[
            pl.BlockSpec((bm, bk), lambda i, k: (i, k)),
            pl.BlockSpec((bk, F), lambda i, k: (k, 0)),
            pl.BlockSpec((F, 16), lambda i, k: (0, 0)),
        ],
        out_specs=[
            pl.BlockSpec((bm, F + XP), lambda i, k: (i, 0)),
            pl.BlockSpec((16, bm), lambda i, k: (0, i)),
        ],
        out_shape=[
            jax.ShapeDtypeStruct((N, F + XP), f32),
            jax.ShapeDtypeStruct((16, N), f32),
        ],
    )(x, w, amat)


def _sc_edge_body(Wd, haug_hbm, s2_hbm, src_hbm, dst_hbm, out_hbm,
                  ssrc_v, sdst_v, si_v, di_v, ex_v, rows_v, acc_sh, sem):
    cid = lax.axis_index("c")
    sid = lax.axis_index("s")
    wid = cid * NS + sid

    pltpu.sync_copy(s2_hbm.at[0], ssrc_v)
    pltpu.sync_copy(s2_hbm.at[1], sdst_v)

    @pl.loop(0, CHUNK)
    def _(r):
        for c in range(Wd // LN):
            rows_v[r, pl.ds(c * LN, LN)] = jnp.zeros((LN,), f32)

    n_rows_per_tile = N // NS
    for t in range(n_rows_per_tile // CHUNK):
        pltpu.sync_copy(
            rows_v, acc_sh.at[pl.ds(sid * n_rows_per_tile + t * CHUNK, CHUNK)])
    plsc.subcore_barrier()

    ebase = wid * EPW

    @pl.loop(0, NCHUNK)
    def _(cc):
        base = ebase + cc * CHUNK
        pltpu.sync_copy(src_hbm.at[pl.ds(base, CHUNK)], si_v)
        pltpu.sync_copy(dst_hbm.at[pl.ds(base, CHUNK)], di_v)

        @pl.loop(0, CHUNK, step=LN)
        def _(j):
            sidx = si_v[pl.ds(j, LN)]
            didx = di_v[pl.ds(j, LN)]
            sv = plsc.load_gather(ssrc_v, [sidx])
            dv = plsc.load_gather(sdst_v, [didx])
            t = sv + dv
            e = jnp.maximum(t, NEG_SLOPE * t)
            ex_v[pl.ds(j, LN)] = jnp.exp(e)

        pltpu.sync_copy(haug_hbm.at[si_v], rows_v)

        @pl.loop(0, CHUNK, step=LN)
        def _(jg):
            exv = ex_v[pl.ds(jg, LN)]
            for jj in range(LN):
                exs = exv[jj]
                for c in range(Wd // LN):
                    sl = pl.ds(c * LN, LN)
                    rows_v[jg + jj, sl] = rows_v[jg + jj, sl] * exs

        pltpu.sync_copy(rows_v, acc_sh.at[di_v], add=True)

    plsc.subcore_barrier()

    for t in range(n_rows_per_tile // CHUNK):
        ro = sid * n_rows_per_tile + t * CHUNK
        pltpu.sync_copy(acc_sh.at[pl.ds(ro, CHUNK)],
                        out_hbm.at[cid].at[pl.ds(ro, CHUNK)])


def sc_edge(haug, s2, src, dst):
    Wd = haug.shape[1]
    cp = pltpu.CompilerParams()
    if "needs_layout_passes" in pltpu.CompilerParams.__dataclass_fields__:
        cp = dataclasses.replace(cp, needs_layout_passes=False)
    if "use_tc_tiling_on_sc" in pltpu.CompilerParams.__dataclass_fields__:
        cp = dataclasses.replace(cp, use_tc_tiling_on_sc=False)
    kern = pl.kernel(
        functools.partial(_sc_edge_body, Wd),
        out_type=jax.ShapeDtypeStruct((NC, N, Wd), f32),
        mesh=plsc.VectorSubcoreMesh(core_axis_name="c", subcore_axis_name="s"),
        scratch_types=[
            pltpu.VMEM((N,), f32),
            pltpu.VMEM((N,), f32),
            pltpu.VMEM((CHUNK,), jnp.int32),
            pltpu.VMEM((CHUNK,), jnp.int32),
            pltpu.VMEM((CHUNK,), f32),
            pltpu.VMEM((CHUNK, Wd), f32),
            pltpu.VMEM_SHARED((N, Wd), f32),
            pltpu.SemaphoreType.DMA,
        ],
        compiler_params=cp,
    )
    return kern(haug, s2, src, dst)


def _finalize_body(F, acc0_ref, acc1_ref, b_ref, h_ref, yn_ref,
                   rn2c_ref, rn2t_ref):
    acc = acc0_ref[0] + acc1_ref[0]
    den = acc[:, F:F + 1] + 1e-16
    out = acc[:, :F] / den + b_ref[...]
    h = jnp.maximum(out, 0.0)
    h_ref[...] = h
    mn = jnp.min(h, axis=1, keepdims=True)
    mx = jnp.max(h, axis=1, keepdims=True)
    yn = (h - mn) / (mx - mn + 1e-12)
    yn_ref[...] = yn
    yn2 = yn * yn
    rn2 = jnp.sum(yn2, axis=1, keepdims=True)
    rn2c_ref[...] = jnp.broadcast_to(rn2, rn2c_ref.shape)
    ones16 = jnp.ones((16, F), f32)
    rn2t_ref[...] = lax.dot_general(
        ones16, yn2, (((1,), (1,)), ((), ())),
        precision=HIGH, preferred_element_type=f32)


def tc_finalize(acc, b):
    F = acc.shape[2] - XP
    bm = 256
    return pl.pallas_call(
        functools.partial(_finalize_body, F),
        grid=(N // bm,),
        in_specs=[
            pl.BlockSpec((1, bm, F + XP), lambda i: (0, i, 0)),
            pl.BlockSpec((1, bm, F + XP), lambda i: (1, i, 0)),
            pl.BlockSpec((1, F), lambda i: (0, 0)),
        ],
        out_specs=[
            pl.BlockSpec((bm, F), lambda i: (i, 0)),
            pl.BlockSpec((bm, F), lambda i: (i, 0)),
            pl.BlockSpec((bm, 8), lambda i: (i, 0)),
            pl.BlockSpec((16, bm), lambda i: (0, i)),
        ],
        out_shape=[
            jax.ShapeDtypeStruct((N, F), f32),
            jax.ShapeDtypeStruct((N, F), f32),
            jax.ShapeDtypeStruct((N, 8), f32),
            jax.ShapeDtypeStruct((16, N), f32),
        ],
    )(acc, acc, b.reshape(1, F))


def _gip_body(HS, bm, att_ref,
              y1i, y1j, y2i, y2j, y3i, y3j,
              r1c, r2c, r3c, r1f, r2f, r3f, r1j, r2j, r3j,
              sim_ref, kf_ref, diagc_ref, minp_ref):
    j = pl.program_id(0)
    i = pl.program_id(1)

    kf = att_ref[0, 3] * sim_ref[...]
    for (yi, yj, rc, rf, rj, g, l) in (
            (y1i, y1j, r1c, r1f, r1j, GAMMAS[0], 0),
            (y2i, y2j, r2c, r2f, r2j, GAMMAS[1], 1),
            (y3i, y3j, r3c, r3f, r3j, GAMMAS[2], 2)):
        c = jnp.sum(rf[0:1, :]) / HS
        dot = lax.dot_general(
            yi[...], yj[...], (((1,), (1,)), ((), ())),
            preferred_element_type=f32)
        dist = (rc[:, 0:1] + rj[0:1, :] - 2.0 * dot) / c
        kf = kf + att_ref[0, l] * jnp.exp(-g * dist)
    kf_ref[...] = kf

    a = jnp.abs(kf)

    @pl.when(i == 0)
    def _():
        diagc_ref[...] = jnp.zeros_like(diagc_ref)

    @pl.when(j == i)
    def _():
        eye = (lax.broadcasted_iota(jnp.int32, (bm, bm), 0) ==
               lax.broadcasted_iota(jnp.int32, (bm, bm), 1))
        dcol = jnp.sum(jnp.where(eye, a, 0.0), axis=1, keepdims=True)
        diagc_ref[...] += jnp.broadcast_to(dcol, diagc_ref.shape)

    BIG = 3.4e38
    posmin = jnp.min(jnp.where(a > 0, a, BIG))

    @pl.when(i == 0)
    def _():
        minp_ref[...] = jnp.full_like(minp_ref, BIG)

    minp_ref[...] = jnp.minimum(minp_ref[...], posmin)


def tc_gip_half(yn1, yn2, yn3, rn2c, rn2t, sim, att, half):
    HS = M_SIZE
    bm = 256
    ng = HS // bm
    lo = half * HS

    def sl(x):
        return x[lo:lo + HS]

    def slt(x):
        return x[:, lo:lo + HS]

    y_specs = []
    y_args = []
    for yn, F in ((yn1, F1), (yn2, F2), (yn3, F3)):
        y_args += [sl(yn), sl(yn)]
        y_specs += [
            pl.BlockSpec((bm, F), lambda j, i: (i, 0)),
            pl.BlockSpec((bm, F), lambda j, i: (j, 0)),
        ]
    r_specs_c = [pl.BlockSpec((bm, 8), lambda j, i: (i, 0))] * 3
    r_specs_f = [pl.BlockSpec((16, HS), lambda j, i: (0, 0))] * 3
    r_specs_j = [pl.BlockSpec((16, bm), lambda j, i: (0, j))] * 3
    rc_args = [sl(rn2c[0]), sl(rn2c[1]), sl(rn2c[2])]
    rt_args = [slt(rn2t[0]), slt(rn2t[1]), slt(rn2t[2])]

    return pl.pallas_call(
        functools.partial(_gip_body, HS, bm),
        grid=(ng, ng),
        in_specs=([pl.BlockSpec(memory_space=pltpu.SMEM)] + y_specs +
                  r_specs_c + r_specs_f + r_specs_j +
                  [pl.BlockSpec((bm, bm), lambda j, i: (i, j))]),
        out_specs=[
            pl.BlockSpec((bm, bm), lambda j, i: (i, j)),
            pl.BlockSpec((bm, 8), lambda j, i: (j, 0)),
            pl.BlockSpec((16, bm), lambda j, i: (0, j)),
        ],
        out_shape=[
            jax.ShapeDtypeStruct((HS, HS), f32),
            jax.ShapeDtypeStruct((HS, 8), f32),
            jax.ShapeDtypeStruct((16, HS), f32),
        ],
    )(att, *y_args, *rc_args, *rt_args, *rt_args, sim)


def _minred_body(x_ref, o_ref):
    o_ref[0, 0] = jnp.min(x_ref[...])


def tc_minreduce(minp):
    return pl.pallas_call(
        _minred_body,
        in_specs=[pl.BlockSpec(minp.shape, lambda: (0, 0))],
        out_specs=pl.BlockSpec(memory_space=pltpu.SMEM),
        out_shape=jax.ShapeDtypeStruct((1, 1), f32),
    )(minp)


def _final_body(nk, mpm_ref, mpd_ref, km_ref, a1_ref, ddm_ref,
                kd_ref, a2_ref, ddd_ref, o_ref):
    k = pl.program_id(2)

    @pl.when(k == 0)
    def _():
        o_ref[...] = jnp.zeros_like(o_ref)

    mpm = mpm_ref[0, 0]
    mpd = mpd_ref[0, 0]

    am = jnp.abs(km_ref[...])
    a2m = jnp.where(am == 0.0, mpm, am)
    ddm = ddm_ref[:, 0:1]
    ddm = jnp.where(ddm == 0.0, mpm, ddm)
    a1s = a1_ref[...] / ddm

    ad = jnp.abs(kd_ref[...])
    a2d = jnp.where(ad == 0.0, mpd, ad)
    ddd = ddd_ref[:, 0:1]
    ddd = jnp.where(ddd == 0.0, mpd, ddd)
    a2s = a2_ref[...] / ddd

    t1 = lax.dot_general(a2m, a1s, (((1,), (0,)), ((), ())),
                         precision=HIGH, preferred_element_type=f32)
    t2 = lax.dot_general(a2s, a2d, (((0,), (1,)), ((), ())),
                         precision=HIGH, preferred_element_type=f32)
    o_ref[...] += 0.5 * (t1 + t2)


def tc_final(km, ddm, mpm, kd, ddd, mpd, alpha1, alpha2):
    bm = 256
    ng = M_SIZE // bm
    return pl.pallas_call(
        functools.partial(_final_body, ng),
        grid=(ng, ng, ng),
        in_specs=[
            pl.BlockSpec(memory_space=pltpu.SMEM),
            pl.BlockSpec(memory_space=pltpu.SMEM),
            pl.BlockSpec((bm, bm), lambda i, j, k: (i, k)),
            pl.BlockSpec((bm, bm), lambda i, j, k: (k, j)),
            pl.BlockSpec((bm, 8), lambda i, j, k: (k, 0)),
            pl.BlockSpec((bm, bm), lambda i, j, k: (j, k)),
            pl.BlockSpec((bm, bm), lambda i, j, k: (k, i)),
            pl.BlockSpec((bm, 8), lambda i, j, k: (k, 0)),
        ],
        out_specs=pl.BlockSpec((bm, bm), lambda i, j, k: (i, j)),
        out_shape=jax.ShapeDtypeStruct((M_SIZE, M_SIZE), f32),
    )(mpm, mpd, km, alpha1, ddm, kd, alpha2, ddd)


def _gat_layer(x, w, a_src, a_dst, b, src, dst):
    haug, s2 = tc_linear(x, w, a_src, a_dst)
    acc = sc_edge(haug, s2, src, dst)
    return tc_finalize(acc, b)


@jax.jit
def kernel(feature, edge_index, mirna_sim, disease_sim,
           W1, a1_src, a1_dst, b1,
           W2, a2_src, a2_dst, b2,
           W3, a3_src, a3_dst, b3,
           att_m, att_d, alpha1, alpha2):
    loop = jnp.arange(N, dtype=edge_index.dtype)
    src = jnp.concatenate([edge_index[0], loop])
    dst = jnp.concatenate([edge_index[1], loop])

    H1, yn1, rc1, rt1 = _gat_layer(feature, W1, a1_src, a1_dst, b1, src, dst)
    H2, yn2, rc2, rt2 = _gat_layer(H1, W2, a2_src, a2_dst, b2, src, dst)
    H3, yn3, rc3, rt3 = _gat_layer(H2, W3, a3_src, a3_dst, b3, src, dst)

    km, dcm, mpm_p = tc_gip_half(yn1, yn2, yn3, (rc1, rc2, rc3),
                                 (rt1, rt2, rt3), mirna_sim, att_m, 0)
    kd, dcd, mpd_p = tc_gip_half(yn1, yn2, yn3, (rc1, rc2, rc3),
                                 (rt1, rt2, rt3), disease_sim, att_d, 1)
    mpm = tc_minreduce(mpm_p)
    mpd = tc_minreduce(mpd_p)
    return tc_final(km, dcm, mpm, kd, dcd, mpd, alpha1, alpha2)


# speedup vs baseline: 18.6492x; 1.0681x over previous
import dataclasses
import functools

import jax
import jax.numpy as jnp
from jax import lax
from jax.experimental import pallas as pl
from jax.experimental.pallas import tpu as pltpu
from jax.experimental.pallas import tpu_sc as plsc

M_SIZE = 2048
D_SIZE = 2048
N = M_SIZE + D_SIZE
E0 = 131072
E = E0 + N
F1, F2, F3 = 128, 64, 32
GAMMAS = (0.03125, 0.03125, 0.03125)
NEG_SLOPE = 0.2

NC, NS, LN = 2, 16, 16
NW = NC * NS
EPW = E // NW
CHUNK = 128
NCHUNK = EPW // CHUNK

XP = 16

f32 = jnp.float32
HIGH = lax.Precision.HIGHEST


def _dot3(a, b, dims):
    bf = jnp.bfloat16
    ah = a.astype(bf)
    al = (a - ah.astype(f32)).astype(bf)
    bh = b.astype(bf)
    bl = (b - bh.astype(f32)).astype(bf)

    def d(x, y):
        return lax.dot_general(x, y, (dims, ((), ())),
                               preferred_element_type=f32)

    return d(ah, bh) + d(ah, bl) + d(al, bh)


def _linear_body(nk, F, x_ref, w_ref, a_ref, haug_ref, s2_ref):
    k = pl.program_id(1)

    @pl.when(k == 0)
    def _():
        haug_ref[...] = jnp.zeros_like(haug_ref)

    h_part = _dot3(x_ref[...], w_ref[...], ((1,), (0,)))
    haug_ref[:, :F] += h_part

    @pl.when(k == nk - 1)
    def _():
        ones_col = jnp.where(
            lax.broadcasted_iota(jnp.int32, (haug_ref.shape[0], XP), 1) == 0,
            1.0, 0.0)
        haug_ref[:, F:] = ones_col
        h_full = haug_ref[:, :F]
        s2_ref[...] = lax.dot_general(
            a_ref[...], h_full, (((0,), (1,)), ((), ())),
            precision=HIGH, preferred_element_type=f32)


def tc_linear(x, w, a_src, a_dst):
    K, F = w.shape
    bm = 256
    bk = min(K, 512)
    nk = K // bk
    amat = jnp.concatenate(
        [a_src[:, None], a_dst[:, None], jnp.zeros((F, 14), f32)], axis=1)
    return pl.pallas_call(
        functools.partial(_linear_body, nk, F),
        grid=(N // bm, nk),
        in_specs=[
            pl.BlockSpec((bm, bk), lambda i, k: (i, k)),
            pl.BlockSpec((bk, F), lambda i, k: (k, 0)),
            pl.BlockSpec((F, 16), lambda i, k: (0, 0)),
        ],
        out_specs=[
            pl.BlockSpec((bm, F + XP), lambda i, k: (i, 0)),
            pl.BlockSpec((16, bm), lambda i, k: (0, i)),
        ],
        out_shape=[
            jax.ShapeDtypeStruct((N, F + XP), f32),
            jax.ShapeDtypeStruct((16, N), f32),
        ],
    )(x, w, amat)


def _sc_edge_body(Wd, haug_hbm, s2_hbm, src_hbm, dst_hbm, out_hbm,
                  ssrc_v, sdst_v, si_v, di_v, ex_v, rows_v, acc_sh, sem):
    cid = lax.axis_index("c")
    sid = lax.axis_index("s")
    wid = cid * NS + sid

    pltpu.sync_copy(s2_hbm.at[0], ssrc_v)
    pltpu.sync_copy(s2_hbm.at[1], sdst_v)

    @pl.loop(0, CHUNK)
    def _(r):
        for c in range(Wd // LN):
            rows_v[r, pl.ds(c * LN, LN)] = jnp.zeros((LN,), f32)

    n_rows_per_tile = N // NS
    for t in range(n_rows_per_tile // CHUNK):
        pltpu.sync_copy(
            rows_v, acc_sh.at[pl.ds(sid * n_rows_per_tile + t * CHUNK, CHUNK)])
    plsc.subcore_barrier()

    ebase = wid * EPW

    @pl.loop(0, NCHUNK)
    def _(cc):
        base = ebase + cc * CHUNK
        pltpu.sync_copy(src_hbm.at[pl.ds(base, CHUNK)], si_v)
        pltpu.sync_copy(dst_hbm.at[pl.ds(base, CHUNK)], di_v)

        @pl.loop(0, CHUNK, step=LN)
        def _(j):
            sidx = si_v[pl.ds(j, LN)]
            didx = di_v[pl.ds(j, LN)]
            sv = plsc.load_gather(ssrc_v, [sidx])
            dv = plsc.load_gather(sdst_v, [didx])
            t = sv + dv
            e = jnp.maximum(t, NEG_SLOPE * t)
            ex_v[pl.ds(j, LN)] = jnp.exp(e)

        pltpu.sync_copy(haug_hbm.at[si_v], rows_v)

        @pl.loop(0, CHUNK, step=LN)
        def _(jg):
            exv = ex_v[pl.ds(jg, LN)]
            for jj in range(LN):
                exs = exv[jj]
                for c in range(Wd // LN):
                    sl = pl.ds(c * LN, LN)
                    rows_v[jg + jj, sl] = rows_v[jg + jj, sl] * exs

        pltpu.sync_copy(rows_v, acc_sh.at[di_v], add=True)

    plsc.subcore_barrier()

    for t in range(n_rows_per_tile // CHUNK):
        ro = sid * n_rows_per_tile + t * CHUNK
        pltpu.sync_copy(acc_sh.at[pl.ds(ro, CHUNK)],
                        out_hbm.at[cid].at[pl.ds(ro, CHUNK)])


def sc_edge(haug, s2, src, dst):
    Wd = haug.shape[1]
    cp = pltpu.CompilerParams()
    if "needs_layout_passes" in pltpu.CompilerParams.__dataclass_fields__:
        cp = dataclasses.replace(cp, needs_layout_passes=False)
    if "use_tc_tiling_on_sc" in pltpu.CompilerParams.__dataclass_fields__:
        cp = dataclasses.replace(cp, use_tc_tiling_on_sc=False)
    kern = pl.kernel(
        functools.partial(_sc_edge_body, Wd),
        out_type=jax.ShapeDtypeStruct((NC, N, Wd), f32),
        mesh=plsc.VectorSubcoreMesh(core_axis_name="c", subcore_axis_name="s"),
        scratch_types=[
            pltpu.VMEM((N,), f32),
            pltpu.VMEM((N,), f32),
            pltpu.VMEM((CHUNK,), jnp.int32),
            pltpu.VMEM((CHUNK,), jnp.int32),
            pltpu.VMEM((CHUNK,), f32),
            pltpu.VMEM((CHUNK, Wd), f32),
            pltpu.VMEM_SHARED((N, Wd), f32),
            pltpu.SemaphoreType.DMA,
        ],
        compiler_params=cp,
    )
    return kern(haug, s2, src, dst)


def _finalize_body(F, acc0_ref, acc1_ref, b_ref, h_ref, yn_ref,
                   rn2c_ref, rn2t_ref):
    acc = acc0_ref[0] + acc1_ref[0]
    den = acc[:, F:F + 1] + 1e-16
    out = acc[:, :F] / den + b_ref[...]
    h = jnp.maximum(out, 0.0)
    h_ref[...] = h
    mn = jnp.min(h, axis=1, keepdims=True)
    mx = jnp.max(h, axis=1, keepdims=True)
    yn = (h - mn) / (mx - mn + 1e-12)
    yn_ref[...] = yn
    yn2 = yn * yn
    rn2 = jnp.sum(yn2, axis=1, keepdims=True)
    rn2c_ref[...] = jnp.broadcast_to(rn2, rn2c_ref.shape)
    ones16 = jnp.ones((16, F), f32)
    rn2t_ref[...] = lax.dot_general(
        ones16, yn2, (((1,), (1,)), ((), ())),
        precision=HIGH, preferred_element_type=f32)


def tc_finalize(acc, b):
    F = acc.shape[2] - XP
    bm = 256
    return pl.pallas_call(
        functools.partial(_finalize_body, F),
        grid=(N // bm,),
        in_specs=[
            pl.BlockSpec((1, bm, F + XP), lambda i: (0, i, 0)),
            pl.BlockSpec((1, bm, F + XP), lambda i: (1, i, 0)),
            pl.BlockSpec((1, F), lambda i: (0, 0)),
        ],
        out_specs=[
            pl.BlockSpec((bm, F), lambda i: (i, 0)),
            pl.BlockSpec((bm, F), lambda i: (i, 0)),
            pl.BlockSpec((bm, 8), lambda i: (i, 0)),
            pl.BlockSpec((16, bm), lambda i: (0, i)),
        ],
        out_shape=[
            jax.ShapeDtypeStruct((N, F), f32),
            jax.ShapeDtypeStruct((N, F), f32),
            jax.ShapeDtypeStruct((N, 8), f32),
            jax.ShapeDtypeStruct((16, N), f32),
        ],
    )(acc, acc, b.reshape(1, F))


def _gip_body(HS, bm, att_ref,
              y1i, y1j, y2i, y2j, y3i, y3j,
              r1c, r2c, r3c, r1f, r2f, r3f, r1j, r2j, r3j,
              sim_ref, kf_ref, diagc_ref, minp_ref):
    j = pl.program_id(0)
    i = pl.program_id(1)

    kf = att_ref[0, 3] * sim_ref[...]
    for (yi, yj, rc, rf, rj, g, l) in (
            (y1i, y1j, r1c, r1f, r1j, GAMMAS[0], 0),
            (y2i, y2j, r2c, r2f, r2j, GAMMAS[1], 1),
            (y3i, y3j, r3c, r3f, r3j, GAMMAS[2], 2)):
        c = jnp.sum(rf[0:1, :]) / HS
        dot = lax.dot_general(
            yi[...], yj[...], (((1,), (1,)), ((), ())),
            preferred_element_type=f32)
        dist = (rc[:, 0:1] + rj[0:1, :] - 2.0 * dot) / c
        kf = kf + att_ref[0, l] * jnp.exp(-g * dist)
    kf_ref[...] = kf

    a = jnp.abs(kf)

    @pl.when(i == 0)
    def _():
        diagc_ref[...] = jnp.zeros_like(diagc_ref)

    @pl.when(j == i)
    def _():
        eye = (lax.broadcasted_iota(jnp.int32, (bm, bm), 0) ==
               lax.broadcasted_iota(jnp.int32, (bm, bm), 1))
        dcol = jnp.sum(jnp.where(eye, a, 0.0), axis=1, keepdims=True)
        diagc_ref[...] += jnp.broadcast_to(dcol, diagc_ref.shape)

    BIG = 3.4e38
    posmin = jnp.min(jnp.where(a > 0, a, BIG))

    @pl.when(i == 0)
    def _():
        minp_ref[...] = jnp.full_like(minp_ref, BIG)

    minp_ref[...] = jnp.minimum(minp_ref[...], posmin)


def tc_gip_half(yn1, yn2, yn3, rn2c, rn2t, sim, att, half):
    HS = M_SIZE
    bm = 256
    ng = HS // bm
    lo = half * HS

    def sl(x):
        return x[lo:lo + HS]

    def slt(x):
        return x[:, lo:lo + HS]

    y_specs = []
    y_args = []
    for yn, F in ((yn1, F1), (yn2, F2), (yn3, F3)):
        y_args += [sl(yn), sl(yn)]
        y_specs += [
            pl.BlockSpec((bm, F), lambda j, i: (i, 0)),
            pl.BlockSpec((bm, F), lambda j, i: (j, 0)),
        ]
    r_specs_c = [pl.BlockSpec((bm, 8), lambda j, i: (i, 0))] * 3
    r_specs_f = [pl.BlockSpec((16, HS), lambda j, i: (0, 0))] * 3
    r_specs_j = [pl.BlockSpec((16, bm), lambda j, i: (0, j))] * 3
    rc_args = [sl(rn2c[0]), sl(rn2c[1]), sl(rn2c[2])]
    rt_args = [slt(rn2t[0]), slt(rn2t[1]), slt(rn2t[2])]

    return pl.pallas_call(
        functools.partial(_gip_body, HS, bm),
        grid=(ng, ng),
        in_specs=([pl.BlockSpec(memory_space=pltpu.SMEM)] + y_specs +
                  r_specs_c + r_specs_f + r_specs_j +
                  [pl.BlockSpec((bm, bm), lambda j, i: (i, j))]),
        out_specs=[
            pl.BlockSpec((bm, bm), lambda j, i: (i, j)),
            pl.BlockSpec((bm, 8), lambda j, i: (j, 0)),
            pl.BlockSpec((16, bm), lambda j, i: (0, j)),
        ],
        out_shape=[
            jax.ShapeDtypeStruct((HS, HS), f32),
            jax.ShapeDtypeStruct((HS, 8), f32),
            jax.ShapeDtypeStruct((16, HS), f32),
        ],
    )(att, *y_args, *rc_args, *rt_args, *rt_args, sim)


def _minred_body(x_ref, o_ref):
    o_ref[0, 0] = jnp.min(x_ref[...])


def tc_minreduce(minp):
    return pl.pallas_call(
        _minred_body,
        in_specs=[pl.BlockSpec(minp.shape, lambda: (0, 0))],
        out_specs=pl.BlockSpec(memory_space=pltpu.SMEM),
        out_shape=jax.ShapeDtypeStruct((1, 1), f32),
    )(minp)


def _final_body(nk, mpm_ref, mpd_ref, km_ref, a1_ref, ddm_ref,
                kd_ref, a2_ref, ddd_ref, o_ref):
    k = pl.program_id(2)

    @pl.when(k == 0)
    def _():
        o_ref[...] = jnp.zeros_like(o_ref)

    mpm = mpm_ref[0, 0]
    mpd = mpd_ref[0, 0]

    am = jnp.abs(km_ref[...])
    a2m = jnp.where(am == 0.0, mpm, am)
    ddm = ddm_ref[:, 0:1]
    ddm = jnp.where(ddm == 0.0, mpm, ddm)
    a1s = a1_ref[...] / ddm

    ad = jnp.abs(kd_ref[...])
    a2d = jnp.where(ad == 0.0, mpd, ad)
    ddd = ddd_ref[:, 0:1]
    ddd = jnp.where(ddd == 0.0, mpd, ddd)
    a2s = a2_ref[...] / ddd

    t1 = _dot3(a2m, a1s, ((1,), (0,)))
    t2 = _dot3(a2s, a2d, ((0,), (1,)))
    o_ref[...] += 0.5 * (t1 + t2)


def tc_final(km, ddm, mpm, kd, ddd, mpd, alpha1, alpha2):
    bm = 256
    ng = M_SIZE // bm
    return pl.pallas_call(
        functools.partial(_final_body, ng),
        grid=(ng, ng, ng),
        in_specs=[
            pl.BlockSpec(memory_space=pltpu.SMEM),
            pl.BlockSpec(memory_space=pltpu.SMEM),
            pl.BlockSpec((bm, bm), lambda i, j, k: (i, k)),
            pl.BlockSpec((bm, bm), lambda i, j, k: (k, j)),
            pl.BlockSpec((bm, 8), lambda i, j, k: (k, 0)),
            pl.BlockSpec((bm, bm), lambda i, j, k: (j, k)),
            pl.BlockSpec((bm, bm), lambda i, j, k: (k, i)),
            pl.BlockSpec((bm, 8), lambda i, j, k: (k, 0)),
        ],
        out_specs=pl.BlockSpec((bm, bm), lambda i, j, k: (i, j)),
        out_shape=jax.ShapeDtypeStruct((M_SIZE, M_SIZE), f32),
    )(mpm, mpd, km, alpha1, ddm, kd, alpha2, ddd)


def _gat_layer(x, w, a_src, a_dst, b, src, dst):
    haug, s2 = tc_linear(x, w, a_src, a_dst)
    acc = sc_edge(haug, s2, src, dst)
    return tc_finalize(acc, b)


@jax.jit
def kernel(feature, edge_index, mirna_sim, disease_sim,
           W1, a1_src, a1_dst, b1,
           W2, a2_src, a2_dst, b2,
           W3, a3_src, a3_dst, b3,
           att_m, att_d, alpha1, alpha2):
    loop = jnp.arange(N, dtype=edge_index.dtype)
    src = jnp.concatenate([edge_index[0], loop])
    dst = jnp.concatenate([edge_index[1], loop])

    H1, yn1, rc1, rt1 = _gat_layer(feature, W1, a1_src, a1_dst, b1, src, dst)
    H2, yn2, rc2, rt2 = _gat_layer(H1, W2, a2_src, a2_dst, b2, src, dst)
    H3, yn3, rc3, rt3 = _gat_layer(H2, W3, a3_src, a3_dst, b3, src, dst)

    km, dcm, mpm_p = tc_gip_half(yn1, yn2, yn3, (rc1, rc2, rc3),
                                 (rt1, rt2, rt3), mirna_sim, att_m, 0)
    kd, dcd, mpd_p = tc_gip_half(yn1, yn2, yn3, (rc1, rc2, rc3),
                                 (rt1, rt2, rt3), disease_sim, att_d, 1)
    mpm = tc_minreduce(mpm_p)
    mpd = tc_minreduce(mpd_p)
    return tc_final(km, dcm, mpm, kd, dcd, mpd, alpha1, alpha2)


# speedup vs baseline: 20.7878x; 1.1147x over previous
import dataclasses
import functools

import jax
import jax.numpy as jnp
from jax import lax
from jax.experimental import pallas as pl
from jax.experimental.pallas import tpu as pltpu
from jax.experimental.pallas import tpu_sc as plsc

M_SIZE = 2048
D_SIZE = 2048
N = M_SIZE + D_SIZE
E0 = 131072
E = E0 + N
F1, F2, F3 = 128, 64, 32
GAMMAS = (0.03125, 0.03125, 0.03125)
NEG_SLOPE = 0.2

NC, NS, LN = 2, 16, 16
NW = NC * NS
EPW = E // NW
CHUNK = 128
NCHUNK = EPW // CHUNK
NBUF = 2

XP = 16

f32 = jnp.float32
HIGH = lax.Precision.HIGHEST


def _dot3(a, b, dims):
    bf = jnp.bfloat16
    ah = a.astype(bf)
    al = (a - ah.astype(f32)).astype(bf)
    bh = b.astype(bf)
    bl = (b - bh.astype(f32)).astype(bf)

    def d(x, y):
        return lax.dot_general(x, y, (dims, ((), ())),
                               preferred_element_type=f32)

    return d(ah, bh) + d(ah, bl) + d(al, bh)


def _linear_body(nk, F, x_ref, w_ref, a_ref, haug_ref, s2_ref):
    k = pl.program_id(1)

    @pl.when(k == 0)
    def _():
        haug_ref[...] = jnp.zeros_like(haug_ref)

    h_part = _dot3(x_ref[...], w_ref[...], ((1,), (0,)))
    haug_ref[:, :F] += h_part

    @pl.when(k == nk - 1)
    def _():
        ones_col = jnp.where(
            lax.broadcasted_iota(jnp.int32, (haug_ref.shape[0], XP), 1) == 0,
            1.0, 0.0)
        haug_ref[:, F:] = ones_col
        h_full = haug_ref[:, :F]
        s2_ref[...] = lax.dot_general(
            a_ref[...], h_full, (((0,), (1,)), ((), ())),
            precision=HIGH, preferred_element_type=f32)


def tc_linear(x, w, a_src, a_dst):
    K, F = w.shape
    bm = 256
    bk = min(K, 512)
    nk = K // bk
    amat = jnp.concatenate(
        [a_src[:, None], a_dst[:, None], jnp.zeros((F, 14), f32)], axis=1)
    return pl.pallas_call(
        functools.partial(_linear_body, nk, F),
        grid=(N // bm, nk),
        in_specs=[
            pl.BlockSpec((bm, bk), lambda i, k: (i, k)),
            pl.BlockSpec((bk, F), lambda i, k: (k, 0)),
            pl.BlockSpec((F, 16), lambda i, k: (0, 0)),
        ],
        out_specs=[
            pl.BlockSpec((bm, F + XP), lambda i, k: (i, 0)),
            pl.BlockSpec((16, bm), lambda i, k: (0, i)),
        ],
        out_shape=[
            jax.ShapeDtypeStruct((N, F + XP), f32),
            jax.ShapeDtypeStruct((16, N), f32),
        ],
    )(x, w, amat)


def _sc_edge_body(Wd, haug_hbm, s2_hbm, src_hbm, dst_hbm, out_hbm,
                  ssrc_v, sdst_v, si_v, di_v, ex_v, *rest):
    bufs = rest[:NBUF]
    acc_sh = rest[NBUF]
    gsems = rest[NBUF + 1:2 * NBUF + 1]
    ssems = rest[2 * NBUF + 1:3 * NBUF + 1]
    cid = lax.axis_index("c")
    sid = lax.axis_index("s")
    wid = cid * NS + sid

    def buf_of(c):
        return (c + NBUF - 1) % NBUF

    pltpu.sync_copy(s2_hbm.at[0], ssrc_v)
    pltpu.sync_copy(s2_hbm.at[1], sdst_v)
    rbase = wid * NCHUNK
    pltpu.sync_copy(src_hbm.at[pl.ds(rbase, NCHUNK)], si_v)
    pltpu.sync_copy(dst_hbm.at[pl.ds(rbase, NCHUNK)], di_v)

    @pl.loop(0, NCHUNK)
    def _(c):
        @pl.loop(0, CHUNK, step=LN)
        def _(j):
            sidx = si_v[c, pl.ds(j, LN)]
            didx = di_v[c, pl.ds(j, LN)]
            sv = plsc.load_gather(ssrc_v, [sidx])
            dv = plsc.load_gather(sdst_v, [didx])
            t = sv + dv
            e = jnp.maximum(t, NEG_SLOPE * t)
            ex_v[c, pl.ds(j, LN)] = jnp.exp(e)

    @pl.loop(0, CHUNK)
    def _(r):
        for c in range(Wd // LN):
            bufs[0][r, pl.ds(c * LN, LN)] = jnp.zeros((LN,), f32)

    n_rows_per_tile = N // NS
    for t in range(n_rows_per_tile // CHUNK):
        pltpu.sync_copy(
            bufs[0],
            acc_sh.at[pl.ds(sid * n_rows_per_tile + t * CHUNK, CHUNK)])

    def gather_start(c, b):
        pltpu.async_copy(haug_hbm.at[si_v.at[c]], bufs[b], gsems[b])

    def gather_wait(c, b):
        pltpu.make_async_copy(haug_hbm.at[si_v.at[c]], bufs[b],
                              gsems[b]).wait()

    def scat_start(c, b):
        pltpu.async_copy(bufs[b], acc_sh.at[di_v.at[c]], ssems[b], add=True)

    def scat_wait(c, b):
        pltpu.make_async_copy(bufs[b], acc_sh.at[di_v.at[c]],
                              ssems[b]).wait()

    def scale(c, b):
        rows = bufs[b]

        @pl.loop(0, CHUNK, step=LN)
        def _(jg):
            exv = ex_v[c, pl.ds(jg, LN)]
            for jj in range(LN):
                exs = exv[jj]
                for cl in range(Wd // LN):
                    sl = pl.ds(cl * LN, LN)
                    rows[jg + jj, sl] = rows[jg + jj, sl] * exs

    lead = NBUF - 1
    for c in range(lead):
        gather_start(c, buf_of(c))
    plsc.subcore_barrier()

    main = ((NCHUNK - 1) // NBUF) * NBUF

    @pl.loop(0, main, step=NBUF)
    def _(cb):
        for b4 in range(NBUF):
            c = cb + b4
            b = (b4 + lead) % NBUF
            gather_wait(c, b)
            scale(c, b)
            scat_start(c, b)
            nb = (b4 + 2 * lead) % NBUF

            @pl.when(c + lead <= NCHUNK - 1)
            def _():
                if b4 == 0:
                    @pl.when(c >= 1)
                    def _():
                        scat_wait(c - 1, nb)
                else:
                    scat_wait(c - 1, nb)
                gather_start(c + lead, nb)

    for c in range(main, NCHUNK):
        gather_wait(c, buf_of(c))
        scale(c, buf_of(c))
        scat_start(c, buf_of(c))
    for c in range(max(0, NCHUNK - NBUF), NCHUNK):
        scat_wait(c, buf_of(c))

    plsc.subcore_barrier()

    for t in range(n_rows_per_tile // CHUNK):
        ro = sid * n_rows_per_tile + t * CHUNK
        pltpu.sync_copy(acc_sh.at[pl.ds(ro, CHUNK)],
                        out_hbm.at[cid].at[pl.ds(ro, CHUNK)])


def sc_edge(haug, s2, src, dst):
    Wd = haug.shape[1]
    cp = pltpu.CompilerParams()
    if "needs_layout_passes" in pltpu.CompilerParams.__dataclass_fields__:
        cp = dataclasses.replace(cp, needs_layout_passes=False)
    if "use_tc_tiling_on_sc" in pltpu.CompilerParams.__dataclass_fields__:
        cp = dataclasses.replace(cp, use_tc_tiling_on_sc=False)
    kern = pl.kernel(
        functools.partial(_sc_edge_body, Wd),
        out_type=jax.ShapeDtypeStruct((NC, N, Wd), f32),
        mesh=plsc.VectorSubcoreMesh(core_axis_name="c", subcore_axis_name="s"),
        scratch_types=[
            pltpu.VMEM((N,), f32),
            pltpu.VMEM((N,), f32),
            pltpu.VMEM((NCHUNK, CHUNK), jnp.int32),
            pltpu.VMEM((NCHUNK, CHUNK), jnp.int32),
            pltpu.VMEM((NCHUNK, CHUNK), f32),
        ] + [pltpu.VMEM((CHUNK, Wd), f32)] * NBUF + [
            pltpu.VMEM_SHARED((N, Wd), f32),
        ] + [pltpu.SemaphoreType.DMA] * (2 * NBUF),
        compiler_params=cp,
    )
    return kern(haug, s2, src.reshape(E // CHUNK, CHUNK),
                dst.reshape(E // CHUNK, CHUNK))


def _finalize_body(F, acc0_ref, acc1_ref, b_ref, h_ref, yn_ref,
                   rn2c_ref, rn2t_ref):
    acc = acc0_ref[0] + acc1_ref[0]
    den = acc[:, F:F + 1] + 1e-16
    out = acc[:, :F] / den + b_ref[...]
    h = jnp.maximum(out, 0.0)
    h_ref[...] = h
    mn = jnp.min(h, axis=1, keepdims=True)
    mx = jnp.max(h, axis=1, keepdims=True)
    yn = (h - mn) / (mx - mn + 1e-12)
    yn_ref[...] = yn
    yn2 = yn * yn
    rn2 = jnp.sum(yn2, axis=1, keepdims=True)
    rn2c_ref[...] = jnp.broadcast_to(rn2, rn2c_ref.shape)
    ones16 = jnp.ones((16, F), f32)
    rn2t_ref[...] = lax.dot_general(
        ones16, yn2, (((1,), (1,)), ((), ())),
        precision=HIGH, preferred_element_type=f32)


def tc_finalize(acc, b):
    F = acc.shape[2] - XP
    bm = 256
    return pl.pallas_call(
        functools.partial(_finalize_body, F),
        grid=(N // bm,),
        in_specs=[
            pl.BlockSpec((1, bm, F + XP), lambda i: (0, i, 0)),
            pl.BlockSpec((1, bm, F + XP), lambda i: (1, i, 0)),
            pl.BlockSpec((1, F), lambda i: (0, 0)),
        ],
        out_specs=[
            pl.BlockSpec((bm, F), lambda i: (i, 0)),
            pl.BlockSpec((bm, F), lambda i: (i, 0)),
            pl.BlockSpec((bm, 8), lambda i: (i, 0)),
            pl.BlockSpec((16, bm), lambda i: (0, i)),
        ],
        out_shape=[
            jax.ShapeDtypeStruct((N, F), f32),
            jax.ShapeDtypeStruct((N, F), f32),
            jax.ShapeDtypeStruct((N, 8), f32),
            jax.ShapeDtypeStruct((16, N), f32),
        ],
    )(acc, acc, b.reshape(1, F))


def _gip_body(HS, bm, att_ref,
              y1i, y1j, y2i, y2j, y3i, y3j,
              r1c, r2c, r3c, r1f, r2f, r3f, r1j, r2j, r3j,
              sim_ref, kf_ref, diagc_ref, minp_ref):
    j = pl.program_id(0)
    i = pl.program_id(1)

    kf = att_ref[0, 3] * sim_ref[...]
    for (yi, yj, rc, rf, rj, g, l) in (
            (y1i, y1j, r1c, r1f, r1j, GAMMAS[0], 0),
            (y2i, y2j, r2c, r2f, r2j, GAMMAS[1], 1),
            (y3i, y3j, r3c, r3f, r3j, GAMMAS[2], 2)):
        c = jnp.sum(rf[0:1, :]) / HS
        dot = lax.dot_general(
            yi[...], yj[...], (((1,), (1,)), ((), ())),
            preferred_element_type=f32)
        dist = (rc[:, 0:1] + rj[0:1, :] - 2.0 * dot) / c
        kf = kf + att_ref[0, l] * jnp.exp(-g * dist)
    kf_ref[...] = kf

    a = jnp.abs(kf)

    @pl.when(i == 0)
    def _():
        diagc_ref[...] = jnp.zeros_like(diagc_ref)

    @pl.when(j == i)
    def _():
        eye = (lax.broadcasted_iota(jnp.int32, (bm, bm), 0) ==
               lax.broadcasted_iota(jnp.int32, (bm, bm), 1))
        dcol = jnp.sum(jnp.where(eye, a, 0.0), axis=1, keepdims=True)
        diagc_ref[...] += jnp.broadcast_to(dcol, diagc_ref.shape)

    BIG = 3.4e38
    posmin = jnp.min(jnp.where(a > 0, a, BIG))

    @pl.when(i == 0)
    def _():
        minp_ref[...] = jnp.full_like(minp_ref, BIG)

    minp_ref[...] = jnp.minimum(minp_ref[...], posmin)


def tc_gip_half(yn1, yn2, yn3, rn2c, rn2t, sim, att, half):
    HS = M_SIZE
    bm = 256
    ng = HS // bm
    lo = half * HS

    def sl(x):
        return x[lo:lo + HS]

    def slt(x):
        return x[:, lo:lo + HS]

    y_specs = []
    y_args = []
    for yn, F in ((yn1, F1), (yn2, F2), (yn3, F3)):
        y_args += [sl(yn), sl(yn)]
        y_specs += [
            pl.BlockSpec((bm, F), lambda j, i: (i, 0)),
            pl.BlockSpec((bm, F), lambda j, i: (j, 0)),
        ]
    r_specs_c = [pl.BlockSpec((bm, 8), lambda j, i: (i, 0))] * 3
    r_specs_f = [pl.BlockSpec((16, HS), lambda j, i: (0, 0))] * 3
    r_specs_j = [pl.BlockSpec((16, bm), lambda j, i: (0, j))] * 3
    rc_args = [sl(rn2c[0]), sl(rn2c[1]), sl(rn2c[2])]
    rt_args = [slt(rn2t[0]), slt(rn2t[1]), slt(rn2t[2])]

    return pl.pallas_call(
        functools.partial(_gip_body, HS, bm),
        grid=(ng, ng),
        in_specs=([pl.BlockSpec(memory_space=pltpu.SMEM)] + y_specs +
                  r_specs_c + r_specs_f + r_specs_j +
                  [pl.BlockSpec((bm, bm), lambda j, i: (i, j))]),
        out_specs=[
            pl.BlockSpec((bm, bm), lambda j, i: (i, j)),
            pl.BlockSpec((bm, 8), lambda j, i: (j, 0)),
            pl.BlockSpec((16, bm), lambda j, i: (0, j)),
        ],
        out_shape=[
            jax.ShapeDtypeStruct((HS, HS), f32),
            jax.ShapeDtypeStruct((HS, 8), f32),
            jax.ShapeDtypeStruct((16, HS), f32),
        ],
    )(att, *y_args, *rc_args, *rt_args, *rt_args, sim)


def _minred_body(x_ref, o_ref):
    o_ref[0, 0] = jnp.min(x_ref[...])


def tc_minreduce(minp):
    return pl.pallas_call(
        _minred_body,
        in_specs=[pl.BlockSpec(minp.shape, lambda: (0, 0))],
        out_specs=pl.BlockSpec(memory_space=pltpu.SMEM),
        out_shape=jax.ShapeDtypeStruct((1, 1), f32),
    )(minp)


def _final_body(nk, mpm_ref, mpd_ref, km_ref, a1_ref, ddm_ref,
                kd_ref, a2_ref, ddd_ref, o_ref):
    k = pl.program_id(2)

    @pl.when(k == 0)
    def _():
        o_ref[...] = jnp.zeros_like(o_ref)

    mpm = mpm_ref[0, 0]
    mpd = mpd_ref[0, 0]

    am = jnp.abs(km_ref[...])
    a2m = jnp.where(am == 0.0, mpm, am)
    ddm = ddm_ref[:, 0:1]
    ddm = jnp.where(ddm == 0.0, mpm, ddm)
    a1s = a1_ref[...] / ddm

    ad = jnp.abs(kd_ref[...])
    a2d = jnp.where(ad == 0.0, mpd, ad)
    ddd = ddd_ref[:, 0:1]
    ddd = jnp.where(ddd == 0.0, mpd, ddd)
    a2s = a2_ref[...] / ddd

    t1 = _dot3(a2m, a1s, ((1,), (0,)))
    t2 = _dot3(a2s, a2d, ((0,), (1,)))
    o_ref[...] += 0.5 * (t1 + t2)


def tc_final(km, ddm, mpm, kd, ddd, mpd, alpha1, alpha2):
    bm = 256
    ng = M_SIZE // bm
    return pl.pallas_call(
        functools.partial(_final_body, ng),
        grid=(ng, ng, ng),
        in_specs=[
            pl.BlockSpec(memory_space=pltpu.SMEM),
            pl.BlockSpec(memory_space=pltpu.SMEM),
            pl.BlockSpec((bm, bm), lambda i, j, k: (i, k)),
            pl.BlockSpec((bm, bm), lambda i, j, k: (k, j)),
            pl.BlockSpec((bm, 8), lambda i, j, k: (k, 0)),
            pl.BlockSpec((bm, bm), lambda i, j, k: (j, k)),
            pl.BlockSpec((bm, bm), lambda i, j, k: (k, i)),
            pl.BlockSpec((bm, 8), lambda i, j, k: (k, 0)),
        ],
        out_specs=pl.BlockSpec((bm, bm), lambda i, j, k: (i, j)),
        out_shape=jax.ShapeDtypeStruct((M_SIZE, M_SIZE), f32),
    )(mpm, mpd, km, alpha1, ddm, kd, alpha2, ddd)


def _gat_layer(x, w, a_src, a_dst, b, src, dst):
    haug, s2 = tc_linear(x, w, a_src, a_dst)
    acc = sc_edge(haug, s2, src, dst)
    return tc_finalize(acc, b)


@jax.jit
def kernel(feature, edge_index, mirna_sim, disease_sim,
           W1, a1_src, a1_dst, b1,
           W2, a2_src, a2_dst, b2,
           W3, a3_src, a3_dst, b3,
           att_m, att_d, alpha1, alpha2):
    loop = jnp.arange(N, dtype=edge_index.dtype)
    src = jnp.concatenate([edge_index[0], loop])
    dst = jnp.concatenate([edge_index[1], loop])

    H1, yn1, rc1, rt1 = _gat_layer(feature, W1, a1_src, a1_dst, b1, src, dst)
    H2, yn2, rc2, rt2 = _gat_layer(H1, W2, a2_src, a2_dst, b2, src, dst)
    H3, yn3, rc3, rt3 = _gat_layer(H2, W3, a3_src, a3_dst, b3, src, dst)

    km, dcm, mpm_p = tc_gip_half(yn1, yn2, yn3, (rc1, rc2, rc3),
                                 (rt1, rt2, rt3), mirna_sim, att_m, 0)
    kd, dcd, mpd_p = tc_gip_half(yn1, yn2, yn3, (rc1, rc2, rc3),
                                 (rt1, rt2, rt3), disease_sim, att_d, 1)
    mpm = tc_minreduce(mpm_p)
    mpd = tc_minreduce(mpd_p)
    return tc_final(km, dcm, mpm, kd, dcd, mpd, alpha1, alpha2)


# speedup vs baseline: 21.9177x; 1.0544x over previous
import dataclasses
import functools

import jax
import jax.numpy as jnp
from jax import lax
from jax.experimental import pallas as pl
from jax.experimental.pallas import tpu as pltpu
from jax.experimental.pallas import tpu_sc as plsc

M_SIZE = 2048
D_SIZE = 2048
N = M_SIZE + D_SIZE
E0 = 131072
E = E0 + N
F1, F2, F3 = 128, 64, 32
GAMMAS = (0.03125, 0.03125, 0.03125)
NEG_SLOPE = 0.2

NC, NS, LN = 2, 16, 16
NW = NC * NS
EPW = E // NW
CHUNK = 128
NCHUNK = EPW // CHUNK
NBUF = 3

XP = 16

f32 = jnp.float32
HIGH = lax.Precision.HIGHEST


def _dot3(a, b, dims):
    bf = jnp.bfloat16
    ah = a.astype(bf)
    al = (a - ah.astype(f32)).astype(bf)
    bh = b.astype(bf)
    bl = (b - bh.astype(f32)).astype(bf)

    def d(x, y):
        return lax.dot_general(x, y, (dims, ((), ())),
                               preferred_element_type=f32)

    return d(ah, bh) + d(ah, bl) + d(al, bh)


def _linear_body(nk, F, x_ref, w_ref, a_ref, haug_ref, s2_ref):
    k = pl.program_id(1)

    @pl.when(k == 0)
    def _():
        haug_ref[...] = jnp.zeros_like(haug_ref)

    h_part = _dot3(x_ref[...], w_ref[...], ((1,), (0,)))
    haug_ref[:, :F] += h_part

    @pl.when(k == nk - 1)
    def _():
        ones_col = jnp.where(
            lax.broadcasted_iota(jnp.int32, (haug_ref.shape[0], XP), 1) == 0,
            1.0, 0.0)
        haug_ref[:, F:] = ones_col
        h_full = haug_ref[:, :F]
        s2_ref[...] = lax.dot_general(
            a_ref[...], h_full, (((0,), (1,)), ((), ())),
            precision=HIGH, preferred_element_type=f32)


def tc_linear(x, w, a_src, a_dst):
    K, F = w.shape
    bm = 256
    bk = min(K, 512)
    nk = K // bk
    amat = jnp.concatenate(
        [a_src[:, None], a_dst[:, None], jnp.zeros((F, 14), f32)], axis=1)
    return pl.pallas_call(
        functools.partial(_linear_body, nk, F),
        grid=(N // bm, nk),
        in_specs=[
            pl.BlockSpec((bm, bk), lambda i, k: (i, k)),
            pl.BlockSpec((bk, F), lambda i, k: (k, 0)),
            pl.BlockSpec((F, 16), lambda i, k: (0, 0)),
        ],
        out_specs=[
            pl.BlockSpec((bm, F + XP), lambda i, k: (i, 0)),
            pl.BlockSpec((16, bm), lambda i, k: (0, i)),
        ],
        out_shape=[
            jax.ShapeDtypeStruct((N, F + XP), f32),
            jax.ShapeDtypeStruct((16, N), f32),
        ],
    )(x, w, amat)


def _sc_edge_body(Wd, haug_hbm, s2_hbm, src_hbm, dst_hbm, out_hbm,
                  ssrc_v, sdst_v, si_v, di_v, ex_v, *rest):
    bufs = rest[:NBUF]
    acc_sh = rest[NBUF]
    gsems = rest[NBUF + 1:2 * NBUF + 1]
    ssems = rest[2 * NBUF + 1:3 * NBUF + 1]
    cid = lax.axis_index("c")
    sid = lax.axis_index("s")
    wid = cid * NS + sid

    def buf_of(c):
        return (c + NBUF - 1) % NBUF

    pltpu.sync_copy(s2_hbm.at[0], ssrc_v)
    pltpu.sync_copy(s2_hbm.at[1], sdst_v)
    rbase = wid * NCHUNK
    pltpu.sync_copy(src_hbm.at[pl.ds(rbase, NCHUNK)], si_v)
    pltpu.sync_copy(dst_hbm.at[pl.ds(rbase, NCHUNK)], di_v)

    @pl.loop(0, NCHUNK)
    def _(c):
        @pl.loop(0, CHUNK, step=LN)
        def _(j):
            sidx = si_v[c, pl.ds(j, LN)]
            didx = di_v[c, pl.ds(j, LN)]
            sv = plsc.load_gather(ssrc_v, [sidx])
            dv = plsc.load_gather(sdst_v, [didx])
            t = sv + dv
            e = jnp.maximum(t, NEG_SLOPE * t)
            ex_v[c, pl.ds(j, LN)] = jnp.exp(e)

    @pl.loop(0, CHUNK)
    def _(r):
        for c in range(Wd // LN):
            bufs[0][r, pl.ds(c * LN, LN)] = jnp.zeros((LN,), f32)

    n_rows_per_tile = N // NS
    for t in range(n_rows_per_tile // CHUNK):
        pltpu.sync_copy(
            bufs[0],
            acc_sh.at[pl.ds(sid * n_rows_per_tile + t * CHUNK, CHUNK)])

    def gather_start(c, b):
        pltpu.async_copy(haug_hbm.at[si_v.at[c]], bufs[b], gsems[b])

    def gather_wait(c, b):
        pltpu.make_async_copy(haug_hbm.at[si_v.at[c]], bufs[b],
                              gsems[b]).wait()

    def scat_start(c, b):
        pltpu.async_copy(bufs[b], acc_sh.at[di_v.at[c]], ssems[b], add=True)

    def scat_wait(c, b):
        pltpu.make_async_copy(bufs[b], acc_sh.at[di_v.at[c]],
                              ssems[b]).wait()

    def scale(c, b):
        rows = bufs[b]

        @pl.loop(0, CHUNK, step=LN)
        def _(jg):
            exv = ex_v[c, pl.ds(jg, LN)]
            for jj in range(LN):
                exs = exv[jj]
                for cl in range(Wd // LN):
                    sl = pl.ds(cl * LN, LN)
                    rows[jg + jj, sl] = rows[jg + jj, sl] * exs

    lead = NBUF - 1
    for c in range(lead):
        gather_start(c, buf_of(c))
    plsc.subcore_barrier()

    main = ((NCHUNK - 1) // NBUF) * NBUF

    @pl.loop(0, main, step=NBUF)
    def _(cb):
        for b4 in range(NBUF):
            c = cb + b4
            b = (b4 + lead) % NBUF
            gather_wait(c, b)
            scale(c, b)
            scat_start(c, b)
            nb = (b4 + 2 * lead) % NBUF

            @pl.when(c + lead <= NCHUNK - 1)
            def _():
                if b4 == 0:
                    @pl.when(c >= 1)
                    def _():
                        scat_wait(c - 1, nb)
                else:
                    scat_wait(c - 1, nb)
                gather_start(c + lead, nb)

    for c in range(main, NCHUNK):
        if c > main - 1 + lead:
            scat_wait(c - NBUF, buf_of(c))
            gather_start(c, buf_of(c))
        gather_wait(c, buf_of(c))
        scale(c, buf_of(c))
        scat_start(c, buf_of(c))
    for c in range(max(0, NCHUNK - NBUF), NCHUNK):
        scat_wait(c, buf_of(c))

    plsc.subcore_barrier()

    for t in range(n_rows_per_tile // CHUNK):
        ro = sid * n_rows_per_tile + t * CHUNK
        pltpu.sync_copy(acc_sh.at[pl.ds(ro, CHUNK)],
                        out_hbm.at[cid].at[pl.ds(ro, CHUNK)])


def sc_edge(haug, s2, src, dst):
    Wd = haug.shape[1]
    cp = pltpu.CompilerParams()
    if "needs_layout_passes" in pltpu.CompilerParams.__dataclass_fields__:
        cp = dataclasses.replace(cp, needs_layout_passes=False)
    if "use_tc_tiling_on_sc" in pltpu.CompilerParams.__dataclass_fields__:
        cp = dataclasses.replace(cp, use_tc_tiling_on_sc=False)
    kern = pl.kernel(
        functools.partial(_sc_edge_body, Wd),
        out_type=jax.ShapeDtypeStruct((NC, N, Wd), f32),
        mesh=plsc.VectorSubcoreMesh(core_axis_name="c", subcore_axis_name="s"),
        scratch_types=[
            pltpu.VMEM((N,), f32),
            pltpu.VMEM((N,), f32),
            pltpu.VMEM((NCHUNK, CHUNK), jnp.int32),
            pltpu.VMEM((NCHUNK, CHUNK), jnp.int32),
            pltpu.VMEM((NCHUNK, CHUNK), f32),
        ] + [pltpu.VMEM((CHUNK, Wd), f32)] * NBUF + [
            pltpu.VMEM_SHARED((N, Wd), f32),
        ] + [pltpu.SemaphoreType.DMA] * (2 * NBUF),
        compiler_params=cp,
    )
    return kern(haug, s2, src.reshape(E // CHUNK, CHUNK),
                dst.reshape(E // CHUNK, CHUNK))


def _finalize_body(F, acc0_ref, acc1_ref, b_ref, h_ref, yn_ref,
                   rn2c_ref, rn2t_ref):
    acc = acc0_ref[0] + acc1_ref[0]
    den = acc[:, F:F + 1] + 1e-16
    out = acc[:, :F] / den + b_ref[...]
    h = jnp.maximum(out, 0.0)
    h_ref[...] = h
    mn = jnp.min(h, axis=1, keepdims=True)
    mx = jnp.max(h, axis=1, keepdims=True)
    yn = (h - mn) / (mx - mn + 1e-12)
    yn_ref[...] = yn
    yn2 = yn * yn
    rn2 = jnp.sum(yn2, axis=1, keepdims=True)
    rn2c_ref[...] = jnp.broadcast_to(rn2, rn2c_ref.shape)
    ones16 = jnp.ones((16, F), f32)
    rn2t_ref[...] = lax.dot_general(
        ones16, yn2, (((1,), (1,)), ((), ())),
        precision=HIGH, preferred_element_type=f32)


def tc_finalize(acc, b):
    F = acc.shape[2] - XP
    bm = 256
    return pl.pallas_call(
        functools.partial(_finalize_body, F),
        grid=(N // bm,),
        in_specs=[
            pl.BlockSpec((1, bm, F + XP), lambda i: (0, i, 0)),
            pl.BlockSpec((1, bm, F + XP), lambda i: (1, i, 0)),
            pl.BlockSpec((1, F), lambda i: (0, 0)),
        ],
        out_specs=[
            pl.BlockSpec((bm, F), lambda i: (i, 0)),
            pl.BlockSpec((bm, F), lambda i: (i, 0)),
            pl.BlockSpec((bm, 8), lambda i: (i, 0)),
            pl.BlockSpec((16, bm), lambda i: (0, i)),
        ],
        out_shape=[
            jax.ShapeDtypeStruct((N, F), f32),
            jax.ShapeDtypeStruct((N, F), f32),
            jax.ShapeDtypeStruct((N, 8), f32),
            jax.ShapeDtypeStruct((16, N), f32),
        ],
    )(acc, acc, b.reshape(1, F))


def _gip_body(HS, bm, att_ref,
              y1i, y1j, y2i, y2j, y3i, y3j,
              r1c, r2c, r3c, r1f, r2f, r3f, r1j, r2j, r3j,
              sim_ref, kf_ref, diagc_ref, minp_ref):
    j = pl.program_id(0)
    i = pl.program_id(1)

    kf = att_ref[0, 3] * sim_ref[...]
    for (yi, yj, rc, rf, rj, g, l) in (
            (y1i, y1j, r1c, r1f, r1j, GAMMAS[0], 0),
            (y2i, y2j, r2c, r2f, r2j, GAMMAS[1], 1),
            (y3i, y3j, r3c, r3f, r3j, GAMMAS[2], 2)):
        c = jnp.sum(rf[0:1, :]) / HS
        dot = lax.dot_general(
            yi[...], yj[...], (((1,), (1,)), ((), ())),
            preferred_element_type=f32)
        dist = (rc[:, 0:1] + rj[0:1, :] - 2.0 * dot) / c
        kf = kf + att_ref[0, l] * jnp.exp(-g * dist)
    kf_ref[...] = kf

    a = jnp.abs(kf)

    @pl.when(i == 0)
    def _():
        diagc_ref[...] = jnp.zeros_like(diagc_ref)

    @pl.when(j == i)
    def _():
        eye = (lax.broadcasted_iota(jnp.int32, (bm, bm), 0) ==
               lax.broadcasted_iota(jnp.int32, (bm, bm), 1))
        dcol = jnp.sum(jnp.where(eye, a, 0.0), axis=1, keepdims=True)
        diagc_ref[...] += jnp.broadcast_to(dcol, diagc_ref.shape)

    BIG = 3.4e38
    posmin = jnp.min(jnp.where(a > 0, a, BIG))

    @pl.when(i == 0)
    def _():
        minp_ref[...] = jnp.full_like(minp_ref, BIG)

    minp_ref[...] = jnp.minimum(minp_ref[...], posmin)


def tc_gip_half(yn1, yn2, yn3, rn2c, rn2t, sim, att, half):
    HS = M_SIZE
    bm = 256
    ng = HS // bm
    lo = half * HS

    def sl(x):
        return x[lo:lo + HS]

    def slt(x):
        return x[:, lo:lo + HS]

    y_specs = []
    y_args = []
    for yn, F in ((yn1, F1), (yn2, F2), (yn3, F3)):
        y_args += [sl(yn), sl(yn)]
        y_specs += [
            pl.BlockSpec((bm, F), lambda j, i: (i, 0)),
            pl.BlockSpec((bm, F), lambda j, i: (j, 0)),
        ]
    r_specs_c = [pl.BlockSpec((bm, 8), lambda j, i: (i, 0))] * 3
    r_specs_f = [pl.BlockSpec((16, HS), lambda j, i: (0, 0))] * 3
    r_specs_j = [pl.BlockSpec((16, bm), lambda j, i: (0, j))] * 3
    rc_args = [sl(rn2c[0]), sl(rn2c[1]), sl(rn2c[2])]
    rt_args = [slt(rn2t[0]), slt(rn2t[1]), slt(rn2t[2])]

    return pl.pallas_call(
        functools.partial(_gip_body, HS, bm),
        grid=(ng, ng),
        in_specs=([pl.BlockSpec(memory_space=pltpu.SMEM)] + y_specs +
                  r_specs_c + r_specs_f + r_specs_j +
                  [pl.BlockSpec((bm, bm), lambda j, i: (i, j))]),
        out_specs=[
            pl.BlockSpec((bm, bm), lambda j, i: (i, j)),
            pl.BlockSpec((bm, 8), lambda j, i: (j, 0)),
            pl.BlockSpec((16, bm), lambda j, i: (0, j)),
        ],
        out_shape=[
            jax.ShapeDtypeStruct((HS, HS), f32),
            jax.ShapeDtypeStruct((HS, 8), f32),
            jax.ShapeDtypeStruct((16, HS), f32),
        ],
    )(att, *y_args, *rc_args, *rt_args, *rt_args, sim)


def _minred_body(x_ref, o_ref):
    o_ref[0, 0] = jnp.min(x_ref[...])


def tc_minreduce(minp):
    return pl.pallas_call(
        _minred_body,
        in_specs=[pl.BlockSpec(minp.shape, lambda: (0, 0))],
        out_specs=pl.BlockSpec(memory_space=pltpu.SMEM),
        out_shape=jax.ShapeDtypeStruct((1, 1), f32),
    )(minp)


def _final_body(nk, mpm_ref, mpd_ref, km_ref, a1_ref, ddm_ref,
                kd_ref, a2_ref, ddd_ref, o_ref):
    k = pl.program_id(2)

    @pl.when(k == 0)
    def _():
        o_ref[...] = jnp.zeros_like(o_ref)

    mpm = mpm_ref[0, 0]
    mpd = mpd_ref[0, 0]

    am = jnp.abs(km_ref[...])
    a2m = jnp.where(am == 0.0, mpm, am)
    ddm = ddm_ref[:, 0:1]
    ddm = jnp.where(ddm == 0.0, mpm, ddm)
    a1s = a1_ref[...] / ddm

    ad = jnp.abs(kd_ref[...])
    a2d = jnp.where(ad == 0.0, mpd, ad)
    ddd = ddd_ref[:, 0:1]
    ddd = jnp.where(ddd == 0.0, mpd, ddd)
    a2s = a2_ref[...] / ddd

    t1 = _dot3(a2m, a1s, ((1,), (0,)))
    t2 = _dot3(a2s, a2d, ((0,), (1,)))
    o_ref[...] += 0.5 * (t1 + t2)


def tc_final(km, ddm, mpm, kd, ddd, mpd, alpha1, alpha2):
    bm = 256
    ng = M_SIZE // bm
    return pl.pallas_call(
        functools.partial(_final_body, ng),
        grid=(ng, ng, ng),
        in_specs=[
            pl.BlockSpec(memory_space=pltpu.SMEM),
            pl.BlockSpec(memory_space=pltpu.SMEM),
            pl.BlockSpec((bm, bm), lambda i, j, k: (i, k)),
            pl.BlockSpec((bm, bm), lambda i, j, k: (k, j)),
            pl.BlockSpec((bm, 8), lambda i, j, k: (k, 0)),
            pl.BlockSpec((bm, bm), lambda i, j, k: (j, k)),
            pl.BlockSpec((bm, bm), lambda i, j, k: (k, i)),
            pl.BlockSpec((bm, 8), lambda i, j, k: (k, 0)),
        ],
        out_specs=pl.BlockSpec((bm, bm), lambda i, j, k: (i, j)),
        out_shape=jax.ShapeDtypeStruct((M_SIZE, M_SIZE), f32),
    )(mpm, mpd, km, alpha1, ddm, kd, alpha2, ddd)


def _gat_layer(x, w, a_src, a_dst, b, src, dst):
    haug, s2 = tc_linear(x, w, a_src, a_dst)
    acc = sc_edge(haug, s2, src, dst)
    return tc_finalize(acc, b)


@jax.jit
def kernel(feature, edge_index, mirna_sim, disease_sim,
           W1, a1_src, a1_dst, b1,
           W2, a2_src, a2_dst, b2,
           W3, a3_src, a3_dst, b3,
           att_m, att_d, alpha1, alpha2):
    loop = jnp.arange(N, dtype=edge_index.dtype)
    src = jnp.concatenate([edge_index[0], loop])
    dst = jnp.concatenate([edge_index[1], loop])

    H1, yn1, rc1, rt1 = _gat_layer(feature, W1, a1_src, a1_dst, b1, src, dst)
    H2, yn2, rc2, rt2 = _gat_layer(H1, W2, a2_src, a2_dst, b2, src, dst)
    H3, yn3, rc3, rt3 = _gat_layer(H2, W3, a3_src, a3_dst, b3, src, dst)

    km, dcm, mpm_p = tc_gip_half(yn1, yn2, yn3, (rc1, rc2, rc3),
                                 (rt1, rt2, rt3), mirna_sim, att_m, 0)
    kd, dcd, mpd_p = tc_gip_half(yn1, yn2, yn3, (rc1, rc2, rc3),
                                 (rt1, rt2, rt3), disease_sim, att_d, 1)
    mpm = tc_minreduce(mpm_p)
    mpd = tc_minreduce(mpd_p)
    return tc_final(km, dcm, mpm, kd, dcd, mpd, alpha1, alpha2)


# speedup vs baseline: 41.1014x; 1.8753x over previous
import dataclasses
import functools

import jax
import jax.numpy as jnp
from jax import lax
from jax.experimental import pallas as pl
from jax.experimental.pallas import tpu as pltpu
from jax.experimental.pallas import tpu_sc as plsc

M_SIZE = 2048
D_SIZE = 2048
N = M_SIZE + D_SIZE
E0 = 131072
E = E0 + N
F1, F2, F3 = 128, 64, 32
GAMMAS = (0.03125, 0.03125, 0.03125)
NEG_SLOPE = 0.2

NC, NS, LN = 2, 16, 16
NW = NC * NS
EPW = E // NW
CHUNK = 128
NCHUNK = EPW // CHUNK
NBUF = 3

XP = 16

f32 = jnp.float32
HIGH = lax.Precision.HIGHEST


def _dot3(a, b, dims):
    bf = jnp.bfloat16
    ah = a.astype(bf)
    al = (a - ah.astype(f32)).astype(bf)
    bh = b.astype(bf)
    bl = (b - bh.astype(f32)).astype(bf)

    def d(x, y):
        return lax.dot_general(x, y, (dims, ((), ())),
                               preferred_element_type=f32)

    return d(ah, bh) + d(ah, bl) + d(al, bh)


def _linear_body(nk, F, x_ref, w_ref, a_ref, haug_ref, s2_ref):
    k = pl.program_id(1)

    @pl.when(k == 0)
    def _():
        haug_ref[...] = jnp.zeros_like(haug_ref)

    h_part = _dot3(x_ref[...], w_ref[...], ((1,), (0,)))
    haug_ref[:, :F] += h_part

    @pl.when(k == nk - 1)
    def _():
        ones_col = jnp.where(
            lax.broadcasted_iota(jnp.int32, (haug_ref.shape[0], XP), 1) == 0,
            1.0, 0.0)
        haug_ref[:, F:] = ones_col
        h_full = haug_ref[:, :F]
        s2_ref[...] = lax.dot_general(
            a_ref[...], h_full, (((0,), (1,)), ((), ())),
            precision=HIGH, preferred_element_type=f32)


def tc_linear(x, w, a_src, a_dst):
    K, F = w.shape
    bm = 512
    bk = min(K, 1024)
    nk = K // bk
    amat = jnp.concatenate(
        [a_src[:, None], a_dst[:, None], jnp.zeros((F, 14), f32)], axis=1)
    return pl.pallas_call(
        functools.partial(_linear_body, nk, F),
        grid=(N // bm, nk),
        in_specs=[
            pl.BlockSpec((bm, bk), lambda i, k: (i, k)),
            pl.BlockSpec((bk, F), lambda i, k: (k, 0)),
            pl.BlockSpec((F, 16), lambda i, k: (0, 0)),
        ],
        out_specs=[
            pl.BlockSpec((bm, F + XP), lambda i, k: (i, 0)),
            pl.BlockSpec((16, bm), lambda i, k: (0, i)),
        ],
        out_shape=[
            jax.ShapeDtypeStruct((N, F + XP), f32),
            jax.ShapeDtypeStruct((16, N), f32),
        ],
    )(x, w, amat)


def _sc_edge_body(Wd, haug_hbm, s2_hbm, src_hbm, dst_hbm, out_hbm,
                  ssrc_v, sdst_v, si_v, di_v, ex_v, *rest):
    bufs = rest[:NBUF]
    acc_sh = rest[NBUF]
    gsems = rest[NBUF + 1:2 * NBUF + 1]
    ssems = rest[2 * NBUF + 1:3 * NBUF + 1]
    cid = lax.axis_index("c")
    sid = lax.axis_index("s")
    wid = cid * NS + sid

    def buf_of(c):
        return (c + NBUF - 1) % NBUF

    pltpu.sync_copy(s2_hbm.at[0], ssrc_v)
    pltpu.sync_copy(s2_hbm.at[1], sdst_v)
    rbase = wid * NCHUNK
    pltpu.sync_copy(src_hbm.at[pl.ds(rbase, NCHUNK)], si_v)
    pltpu.sync_copy(dst_hbm.at[pl.ds(rbase, NCHUNK)], di_v)

    @pl.loop(0, NCHUNK)
    def _(c):
        @pl.loop(0, CHUNK, step=LN)
        def _(j):
            sidx = si_v[c, pl.ds(j, LN)]
            didx = di_v[c, pl.ds(j, LN)]
            sv = plsc.load_gather(ssrc_v, [sidx])
            dv = plsc.load_gather(sdst_v, [didx])
            t = sv + dv
            e = jnp.maximum(t, NEG_SLOPE * t)
            ex_v[c, pl.ds(j, LN)] = jnp.exp(e)

    @pl.loop(0, CHUNK)
    def _(r):
        for c in range(Wd // LN):
            bufs[0][r, pl.ds(c * LN, LN)] = jnp.zeros((LN,), f32)

    n_rows_per_tile = N // NS
    for t in range(n_rows_per_tile // CHUNK):
        pltpu.sync_copy(
            bufs[0],
            acc_sh.at[pl.ds(sid * n_rows_per_tile + t * CHUNK, CHUNK)])

    def gather_start(c, b):
        pltpu.async_copy(haug_hbm.at[si_v.at[c]], bufs[b], gsems[b])

    def gather_wait(c, b):
        pltpu.make_async_copy(haug_hbm.at[si_v.at[c]], bufs[b],
                              gsems[b]).wait()

    def scat_start(c, b):
        pltpu.async_copy(bufs[b], acc_sh.at[di_v.at[c]], ssems[b], add=True)

    def scat_wait(c, b):
        pltpu.make_async_copy(bufs[b], acc_sh.at[di_v.at[c]],
                              ssems[b]).wait()

    def scale(c, b):
        rows = bufs[b]

        @pl.loop(0, CHUNK, step=LN)
        def _(jg):
            exv = ex_v[c, pl.ds(jg, LN)]
            for jj in range(LN):
                exs = exv[jj]
                for cl in range(Wd // LN):
                    sl = pl.ds(cl * LN, LN)
                    rows[jg + jj, sl] = rows[jg + jj, sl] * exs

    lead = NBUF - 1
    for c in range(lead):
        gather_start(c, buf_of(c))
    plsc.subcore_barrier()

    main = ((NCHUNK - 1) // NBUF) * NBUF

    @pl.loop(0, main, step=NBUF)
    def _(cb):
        for b4 in range(NBUF):
            c = cb + b4
            b = (b4 + lead) % NBUF
            gather_wait(c, b)
            scale(c, b)
            scat_start(c, b)
            nb = (b4 + 2 * lead) % NBUF

            @pl.when(c + lead <= NCHUNK - 1)
            def _():
                if b4 == 0:
                    @pl.when(c >= 1)
                    def _():
                        scat_wait(c - 1, nb)
                else:
                    scat_wait(c - 1, nb)
                gather_start(c + lead, nb)

    for c in range(main, NCHUNK):
        if c > main - 1 + lead:
            scat_wait(c - NBUF, buf_of(c))
            gather_start(c, buf_of(c))
        gather_wait(c, buf_of(c))
        scale(c, buf_of(c))
        scat_start(c, buf_of(c))
    for c in range(max(0, NCHUNK - NBUF), NCHUNK):
        scat_wait(c, buf_of(c))

    plsc.subcore_barrier()

    for t in range(n_rows_per_tile // CHUNK):
        ro = sid * n_rows_per_tile + t * CHUNK
        pltpu.sync_copy(acc_sh.at[pl.ds(ro, CHUNK)],
                        out_hbm.at[cid].at[pl.ds(ro, CHUNK)])


def sc_edge(haug, s2, src, dst):
    Wd = haug.shape[1]
    cp = pltpu.CompilerParams()
    if "needs_layout_passes" in pltpu.CompilerParams.__dataclass_fields__:
        cp = dataclasses.replace(cp, needs_layout_passes=False)
    if "use_tc_tiling_on_sc" in pltpu.CompilerParams.__dataclass_fields__:
        cp = dataclasses.replace(cp, use_tc_tiling_on_sc=False)
    kern = pl.kernel(
        functools.partial(_sc_edge_body, Wd),
        out_type=jax.ShapeDtypeStruct((NC, N, Wd), f32),
        mesh=plsc.VectorSubcoreMesh(core_axis_name="c", subcore_axis_name="s"),
        scratch_types=[
            pltpu.VMEM((N,), f32),
            pltpu.VMEM((N,), f32),
            pltpu.VMEM((NCHUNK, CHUNK), jnp.int32),
            pltpu.VMEM((NCHUNK, CHUNK), jnp.int32),
            pltpu.VMEM((NCHUNK, CHUNK), f32),
        ] + [pltpu.VMEM((CHUNK, Wd), f32)] * NBUF + [
            pltpu.VMEM_SHARED((N, Wd), f32),
        ] + [pltpu.SemaphoreType.DMA] * (2 * NBUF),
        compiler_params=cp,
    )
    return kern(haug, s2, src.reshape(E // CHUNK, CHUNK),
                dst.reshape(E // CHUNK, CHUNK))


def _finalize_body(F, acc0_ref, acc1_ref, b_ref, h_ref, yn_ref,
                   rn2c_ref, rn2t_ref):
    acc = acc0_ref[0] + acc1_ref[0]
    den = acc[:, F:F + 1] + 1e-16
    out = acc[:, :F] / den + b_ref[...]
    h = jnp.maximum(out, 0.0)
    h_ref[...] = h
    mn = jnp.min(h, axis=1, keepdims=True)
    mx = jnp.max(h, axis=1, keepdims=True)
    yn = (h - mn) / (mx - mn + 1e-12)
    yn_ref[...] = yn
    yn2 = yn * yn
    rn2 = jnp.sum(yn2, axis=1, keepdims=True)
    rn2c_ref[...] = jnp.broadcast_to(rn2, rn2c_ref.shape)
    ones16 = jnp.ones((16, F), f32)
    rn2t_ref[...] = lax.dot_general(
        ones16, yn2, (((1,), (1,)), ((), ())),
        precision=HIGH, preferred_element_type=f32)


def tc_finalize(acc, b):
    F = acc.shape[2] - XP
    bm = 256
    return pl.pallas_call(
        functools.partial(_finalize_body, F),
        grid=(N // bm,),
        in_specs=[
            pl.BlockSpec((1, bm, F + XP), lambda i: (0, i, 0)),
            pl.BlockSpec((1, bm, F + XP), lambda i: (1, i, 0)),
            pl.BlockSpec((1, F), lambda i: (0, 0)),
        ],
        out_specs=[
            pl.BlockSpec((bm, F), lambda i: (i, 0)),
            pl.BlockSpec((bm, F), lambda i: (i, 0)),
            pl.BlockSpec((bm, 8), lambda i: (i, 0)),
            pl.BlockSpec((16, bm), lambda i: (0, i)),
        ],
        out_shape=[
            jax.ShapeDtypeStruct((N, F), f32),
            jax.ShapeDtypeStruct((N, F), f32),
            jax.ShapeDtypeStruct((N, 8), f32),
            jax.ShapeDtypeStruct((16, N), f32),
        ],
    )(acc, acc, b.reshape(1, F))


def _gip_body(HS, bm, att_ref,
              y1i, y1j, y2i, y2j, y3i, y3j,
              r1c, r2c, r3c, r1f, r2f, r3f, r1j, r2j, r3j,
              sim_ref, kf_ref, diagc_ref, minp_ref):
    j = pl.program_id(0)
    i = pl.program_id(1)

    kf = att_ref[0, 3] * sim_ref[...]
    for (yi, yj, rc, rf, rj, g, l) in (
            (y1i, y1j, r1c, r1f, r1j, GAMMAS[0], 0),
            (y2i, y2j, r2c, r2f, r2j, GAMMAS[1], 1),
            (y3i, y3j, r3c, r3f, r3j, GAMMAS[2], 2)):
        c = jnp.sum(rf[0:1, :]) / HS
        dot = lax.dot_general(
            yi[...], yj[...], (((1,), (1,)), ((), ())),
            preferred_element_type=f32)
        dist = (rc[:, 0:1] + rj[0:1, :] - 2.0 * dot) / c
        kf = kf + att_ref[0, l] * jnp.exp(-g * dist)
    kf_ref[...] = kf

    a = jnp.abs(kf)

    @pl.when(i == 0)
    def _():
        diagc_ref[...] = jnp.zeros_like(diagc_ref)

    @pl.when(j == i)
    def _():
        eye = (lax.broadcasted_iota(jnp.int32, (bm, bm), 0) ==
               lax.broadcasted_iota(jnp.int32, (bm, bm), 1))
        dcol = jnp.sum(jnp.where(eye, a, 0.0), axis=1, keepdims=True)
        diagc_ref[...] += jnp.broadcast_to(dcol, diagc_ref.shape)

    BIG = 3.4e38
    posmin = jnp.min(jnp.where(a > 0, a, BIG))

    @pl.when(i == 0)
    def _():
        minp_ref[...] = jnp.full_like(minp_ref, BIG)

    minp_ref[...] = jnp.minimum(minp_ref[...], posmin)


def tc_gip_half(yn1, yn2, yn3, rn2c, rn2t, sim, att, half):
    HS = M_SIZE
    bm = 512
    ng = HS // bm
    lo = half * HS

    def sl(x):
        return x[lo:lo + HS]

    def slt(x):
        return x[:, lo:lo + HS]

    y_specs = []
    y_args = []
    for yn, F in ((yn1, F1), (yn2, F2), (yn3, F3)):
        y_args += [sl(yn), sl(yn)]
        y_specs += [
            pl.BlockSpec((bm, F), lambda j, i: (i, 0)),
            pl.BlockSpec((bm, F), lambda j, i: (j, 0)),
        ]
    r_specs_c = [pl.BlockSpec((bm, 8), lambda j, i: (i, 0))] * 3
    r_specs_f = [pl.BlockSpec((16, HS), lambda j, i: (0, 0))] * 3
    r_specs_j = [pl.BlockSpec((16, bm), lambda j, i: (0, j))] * 3
    rc_args = [sl(rn2c[0]), sl(rn2c[1]), sl(rn2c[2])]
    rt_args = [slt(rn2t[0]), slt(rn2t[1]), slt(rn2t[2])]

    return pl.pallas_call(
        functools.partial(_gip_body, HS, bm),
        grid=(ng, ng),
        in_specs=([pl.BlockSpec(memory_space=pltpu.SMEM)] + y_specs +
                  r_specs_c + r_specs_f + r_specs_j +
                  [pl.BlockSpec((bm, bm), lambda j, i: (i, j))]),
        out_specs=[
            pl.BlockSpec((bm, bm), lambda j, i: (i, j)),
            pl.BlockSpec((bm, 8), lambda j, i: (j, 0)),
            pl.BlockSpec((16, bm), lambda j, i: (0, j)),
        ],
        out_shape=[
            jax.ShapeDtypeStruct((HS, HS), f32),
            jax.ShapeDtypeStruct((HS, 8), f32),
            jax.ShapeDtypeStruct((16, HS), f32),
        ],
    )(att, *y_args, *rc_args, *rt_args, *rt_args, sim)


def _minred_body(x_ref, o_ref):
    o_ref[0, 0] = jnp.min(x_ref[...])


def tc_minreduce(minp):
    return pl.pallas_call(
        _minred_body,
        in_specs=[pl.BlockSpec(minp.shape, lambda: (0, 0))],
        out_specs=pl.BlockSpec(memory_space=pltpu.SMEM),
        out_shape=jax.ShapeDtypeStruct((1, 1), f32),
    )(minp)


def _final_body(nk, mpm_ref, mpd_ref, km_ref, a1_ref, ddm_ref,
                kd_ref, a2_ref, ddd_ref, o_ref):
    k = pl.program_id(2)

    @pl.when(k == 0)
    def _():
        o_ref[...] = jnp.zeros_like(o_ref)

    mpm = mpm_ref[0, 0]
    mpd = mpd_ref[0, 0]

    am = jnp.abs(km_ref[...])
    a2m = jnp.where(am == 0.0, mpm, am)
    ddm = ddm_ref[:, 0:1]
    ddm = jnp.where(ddm == 0.0, mpm, ddm)
    a1s = a1_ref[...] / ddm

    ad = jnp.abs(kd_ref[...])
    a2d = jnp.where(ad == 0.0, mpd, ad)
    ddd = ddd_ref[:, 0:1]
    ddd = jnp.where(ddd == 0.0, mpd, ddd)
    a2s = a2_ref[...] / ddd

    t1 = _dot3(a2m, a1s, ((1,), (0,)))
    t2 = _dot3(a2s, a2d, ((0,), (1,)))
    o_ref[...] += 0.5 * (t1 + t2)


def tc_final(km, ddm, mpm, kd, ddd, mpd, alpha1, alpha2):
    bm = 512
    ng = M_SIZE // bm
    return pl.pallas_call(
        functools.partial(_final_body, ng),
        grid=(ng, ng, ng),
        in_specs=[
            pl.BlockSpec(memory_space=pltpu.SMEM),
            pl.BlockSpec(memory_space=pltpu.SMEM),
            pl.BlockSpec((bm, bm), lambda i, j, k: (i, k)),
            pl.BlockSpec((bm, bm), lambda i, j, k: (k, j)),
            pl.BlockSpec((bm, 8), lambda i, j, k: (k, 0)),
            pl.BlockSpec((bm, bm), lambda i, j, k: (j, k)),
            pl.BlockSpec((bm, bm), lambda i, j, k: (k, i)),
            pl.BlockSpec((bm, 8), lambda i, j, k: (k, 0)),
        ],
        out_specs=pl.BlockSpec((bm, bm), lambda i, j, k: (i, j)),
        out_shape=jax.ShapeDtypeStruct((M_SIZE, M_SIZE), f32),
    )(mpm, mpd, km, alpha1, ddm, kd, alpha2, ddd)


def _gat_layer(x, w, a_src, a_dst, b, src, dst):
    haug, s2 = tc_linear(x, w, a_src, a_dst)
    acc = sc_edge(haug, s2, src, dst)
    return tc_finalize(acc, b)


@jax.jit
def kernel(feature, edge_index, mirna_sim, disease_sim,
           W1, a1_src, a1_dst, b1,
           W2, a2_src, a2_dst, b2,
           W3, a3_src, a3_dst, b3,
           att_m, att_d, alpha1, alpha2):
    loop = jnp.arange(N, dtype=edge_index.dtype)
    src = jnp.concatenate([edge_index[0], loop])
    dst = jnp.concatenate([edge_index[1], loop])

    H1, yn1, rc1, rt1 = _gat_layer(feature, W1, a1_src, a1_dst, b1, src, dst)
    H2, yn2, rc2, rt2 = _gat_layer(H1, W2, a2_src, a2_dst, b2, src, dst)
    H3, yn3, rc3, rt3 = _gat_layer(H2, W3, a3_src, a3_dst, b3, src, dst)

    km, dcm, mpm_p = tc_gip_half(yn1, yn2, yn3, (rc1, rc2, rc3),
                                 (rt1, rt2, rt3), mirna_sim, att_m, 0)
    kd, dcd, mpd_p = tc_gip_half(yn1, yn2, yn3, (rc1, rc2, rc3),
                                 (rt1, rt2, rt3), disease_sim, att_d, 1)
    mpm = tc_minreduce(mpm_p)
    mpd = tc_minreduce(mpd_p)
    return tc_final(km, dcm, mpm, kd, dcd, mpd, alpha1, alpha2)


# speedup vs baseline: 43.4333x; 1.0567x over previous
import dataclasses
import functools

import jax
import jax.numpy as jnp
from jax import lax
from jax.experimental import pallas as pl
from jax.experimental.pallas import tpu as pltpu
from jax.experimental.pallas import tpu_sc as plsc

M_SIZE = 2048
D_SIZE = 2048
N = M_SIZE + D_SIZE
E0 = 131072
E = E0 + N
F1, F2, F3 = 128, 64, 32
GAMMAS = (0.03125, 0.03125, 0.03125)
NEG_SLOPE = 0.2

NC, NS, LN = 2, 16, 16
NW = NC * NS
EPW = E // NW
CHUNK = 128
NCHUNK = EPW // CHUNK
NBUF = 3

XP = 16

f32 = jnp.float32
HIGH = lax.Precision.HIGHEST


def _dot3(a, b, dims):
    bf = jnp.bfloat16
    ah = a.astype(bf)
    al = (a - ah.astype(f32)).astype(bf)
    bh = b.astype(bf)
    bl = (b - bh.astype(f32)).astype(bf)

    def d(x, y):
        return lax.dot_general(x, y, (dims, ((), ())),
                               preferred_element_type=f32)

    return d(ah, bh) + d(ah, bl) + d(al, bh)


def _linear_body(nk, F, x_ref, w_ref, a_ref, haug_ref, s2_ref):
    k = pl.program_id(1)

    @pl.when(k == 0)
    def _():
        haug_ref[...] = jnp.zeros_like(haug_ref)

    h_part = _dot3(x_ref[...], w_ref[...], ((1,), (0,)))
    haug_ref[:, :F] += h_part

    @pl.when(k == nk - 1)
    def _():
        ones_col = jnp.where(
            lax.broadcasted_iota(jnp.int32, (haug_ref.shape[0], XP), 1) == 0,
            1.0, 0.0)
        haug_ref[:, F:] = ones_col
        h_full = haug_ref[:, :F]
        s2_ref[...] = lax.dot_general(
            a_ref[...], h_full, (((0,), (1,)), ((), ())),
            precision=HIGH, preferred_element_type=f32)


def tc_linear(x, w, a_src, a_dst):
    K, F = w.shape
    bm = 512
    bk = min(K, 1024)
    nk = K // bk
    amat = jnp.concatenate(
        [a_src[:, None], a_dst[:, None], jnp.zeros((F, 14), f32)], axis=1)
    return pl.pallas_call(
        functools.partial(_linear_body, nk, F),
        grid=(N // bm, nk),
        in_specs=[
            pl.BlockSpec((bm, bk), lambda i, k: (i, k)),
            pl.BlockSpec((bk, F), lambda i, k: (k, 0)),
            pl.BlockSpec((F, 16), lambda i, k: (0, 0)),
        ],
        out_specs=[
            pl.BlockSpec((bm, F + XP), lambda i, k: (i, 0)),
            pl.BlockSpec((16, bm), lambda i, k: (0, i)),
        ],
        out_shape=[
            jax.ShapeDtypeStruct((N, F + XP), f32),
            jax.ShapeDtypeStruct((16, N), f32),
        ],
    )(x, w, amat)


def _sc_edge_body(Wd, haug_hbm, s2_hbm, src_hbm, dst_hbm, out_hbm,
                  ssrc_v, sdst_v, si_v, di_v, ex_v, *rest):
    bufs = rest[:NBUF]
    acc_sh = rest[NBUF]
    gsems = rest[NBUF + 1:2 * NBUF + 1]
    ssems = rest[2 * NBUF + 1:3 * NBUF + 1]
    cid = lax.axis_index("c")
    sid = lax.axis_index("s")
    wid = cid * NS + sid

    def buf_of(c):
        return (c + NBUF - 1) % NBUF

    pltpu.sync_copy(s2_hbm.at[0], ssrc_v)
    pltpu.sync_copy(s2_hbm.at[1], sdst_v)
    rbase = wid * NCHUNK
    pltpu.sync_copy(src_hbm.at[pl.ds(rbase, NCHUNK)], si_v)
    pltpu.sync_copy(dst_hbm.at[pl.ds(rbase, NCHUNK)], di_v)

    @pl.loop(0, NCHUNK)
    def _(c):
        @pl.loop(0, CHUNK, step=LN)
        def _(j):
            sidx = si_v[c, pl.ds(j, LN)]
            didx = di_v[c, pl.ds(j, LN)]
            sv = plsc.load_gather(ssrc_v, [sidx])
            dv = plsc.load_gather(sdst_v, [didx])
            t = sv + dv
            e = jnp.maximum(t, NEG_SLOPE * t)
            ex_v[c, pl.ds(j, LN)] = jnp.exp(e)

    @pl.loop(0, CHUNK)
    def _(r):
        for c in range(Wd // LN):
            bufs[0][r, pl.ds(c * LN, LN)] = jnp.zeros((LN,), f32)

    n_rows_per_tile = N // NS
    for t in range(n_rows_per_tile // CHUNK):
        pltpu.sync_copy(
            bufs[0],
            acc_sh.at[pl.ds(sid * n_rows_per_tile + t * CHUNK, CHUNK)])

    def gather_start(c, b):
        pltpu.async_copy(haug_hbm.at[si_v.at[c]], bufs[b], gsems[b])

    def gather_wait(c, b):
        pltpu.make_async_copy(haug_hbm.at[si_v.at[c]], bufs[b],
                              gsems[b]).wait()

    def scat_start(c, b):
        pltpu.async_copy(bufs[b], acc_sh.at[di_v.at[c]], ssems[b], add=True)

    def scat_wait(c, b):
        pltpu.make_async_copy(bufs[b], acc_sh.at[di_v.at[c]],
                              ssems[b]).wait()

    def scale(c, b):
        rows = bufs[b]

        @pl.loop(0, CHUNK, step=LN)
        def _(jg):
            exv = ex_v[c, pl.ds(jg, LN)]
            for jj in range(LN):
                exs = exv[jj]
                for cl in range(Wd // LN):
                    sl = pl.ds(cl * LN, LN)
                    rows[jg + jj, sl] = rows[jg + jj, sl] * exs

    lead = NBUF - 1
    for c in range(lead):
        gather_start(c, buf_of(c))
    plsc.subcore_barrier()

    main = ((NCHUNK - 1) // NBUF) * NBUF

    @pl.loop(0, main, step=NBUF)
    def _(cb):
        for b4 in range(NBUF):
            c = cb + b4
            b = (b4 + lead) % NBUF
            gather_wait(c, b)
            scale(c, b)
            scat_start(c, b)
            nb = (b4 + 2 * lead) % NBUF

            @pl.when(c + lead <= NCHUNK - 1)
            def _():
                if b4 == 0:
                    @pl.when(c >= 1)
                    def _():
                        scat_wait(c - 1, nb)
                else:
                    scat_wait(c - 1, nb)
                gather_start(c + lead, nb)

    for c in range(main, NCHUNK):
        if c > main - 1 + lead:
            scat_wait(c - NBUF, buf_of(c))
            gather_start(c, buf_of(c))
        gather_wait(c, buf_of(c))
        scale(c, buf_of(c))
        scat_start(c, buf_of(c))
    for c in range(max(0, NCHUNK - NBUF), NCHUNK):
        scat_wait(c, buf_of(c))

    plsc.subcore_barrier()

    for t in range(n_rows_per_tile // CHUNK):
        ro = sid * n_rows_per_tile + t * CHUNK
        pltpu.sync_copy(acc_sh.at[pl.ds(ro, CHUNK)],
                        out_hbm.at[cid].at[pl.ds(ro, CHUNK)])


def sc_edge(haug, s2, src, dst):
    Wd = haug.shape[1]
    cp = pltpu.CompilerParams()
    if "needs_layout_passes" in pltpu.CompilerParams.__dataclass_fields__:
        cp = dataclasses.replace(cp, needs_layout_passes=False)
    if "use_tc_tiling_on_sc" in pltpu.CompilerParams.__dataclass_fields__:
        cp = dataclasses.replace(cp, use_tc_tiling_on_sc=False)
    kern = pl.kernel(
        functools.partial(_sc_edge_body, Wd),
        out_type=jax.ShapeDtypeStruct((NC, N, Wd), f32),
        mesh=plsc.VectorSubcoreMesh(core_axis_name="c", subcore_axis_name="s"),
        scratch_types=[
            pltpu.VMEM((N,), f32),
            pltpu.VMEM((N,), f32),
            pltpu.VMEM((NCHUNK, CHUNK), jnp.int32),
            pltpu.VMEM((NCHUNK, CHUNK), jnp.int32),
            pltpu.VMEM((NCHUNK, CHUNK), f32),
        ] + [pltpu.VMEM((CHUNK, Wd), f32)] * NBUF + [
            pltpu.VMEM_SHARED((N, Wd), f32),
        ] + [pltpu.SemaphoreType.DMA] * (2 * NBUF),
        compiler_params=cp,
    )
    return kern(haug, s2, src.reshape(E // CHUNK, CHUNK),
                dst.reshape(E // CHUNK, CHUNK))


def _finalize_body(F, acc0_ref, acc1_ref, b_ref, h_ref, yn_ref,
                   rn2c_ref, rn2t_ref):
    acc = acc0_ref[0] + acc1_ref[0]
    den = acc[:, F:F + 1] + 1e-16
    out = acc[:, :F] / den + b_ref[...]
    h = jnp.maximum(out, 0.0)
    h_ref[...] = h
    mn = jnp.min(h, axis=1, keepdims=True)
    mx = jnp.max(h, axis=1, keepdims=True)
    yn = (h - mn) / (mx - mn + 1e-12)
    yn_ref[...] = yn
    yn2 = yn * yn
    rn2 = jnp.sum(yn2, axis=1, keepdims=True)
    rn2c_ref[...] = jnp.broadcast_to(rn2, rn2c_ref.shape)
    ones16 = jnp.ones((16, F), f32)
    rn2t_ref[...] = lax.dot_general(
        ones16, yn2, (((1,), (1,)), ((), ())),
        precision=HIGH, preferred_element_type=f32)


def tc_finalize(acc, b):
    F = acc.shape[2] - XP
    bm = 256
    return pl.pallas_call(
        functools.partial(_finalize_body, F),
        grid=(N // bm,),
        in_specs=[
            pl.BlockSpec((1, bm, F + XP), lambda i: (0, i, 0)),
            pl.BlockSpec((1, bm, F + XP), lambda i: (1, i, 0)),
            pl.BlockSpec((1, F), lambda i: (0, 0)),
        ],
        out_specs=[
            pl.BlockSpec((bm, F), lambda i: (i, 0)),
            pl.BlockSpec((bm, F), lambda i: (i, 0)),
            pl.BlockSpec((bm, 8), lambda i: (i, 0)),
            pl.BlockSpec((16, bm), lambda i: (0, i)),
        ],
        out_shape=[
            jax.ShapeDtypeStruct((N, F), f32),
            jax.ShapeDtypeStruct((N, F), f32),
            jax.ShapeDtypeStruct((N, 8), f32),
            jax.ShapeDtypeStruct((16, N), f32),
        ],
    )(acc, acc, b.reshape(1, F))


def _gip_body(HS, bm, att_ref,
              y1i, y1j, y2i, y2j, y3i, y3j,
              r1c, r2c, r3c, r1f, r2f, r3f, r1j, r2j, r3j,
              sim_ref, kf_ref, diagc_ref, minp_ref):
    j = pl.program_id(0)
    i = pl.program_id(1)

    kf = att_ref[0, 3] * sim_ref[...]
    for (yi, yj, rc, rf, rj, g, l) in (
            (y1i, y1j, r1c, r1f, r1j, GAMMAS[0], 0),
            (y2i, y2j, r2c, r2f, r2j, GAMMAS[1], 1),
            (y3i, y3j, r3c, r3f, r3j, GAMMAS[2], 2)):
        c = jnp.sum(rf[0:1, :]) / HS
        dot = lax.dot_general(
            yi[...], yj[...], (((1,), (1,)), ((), ())),
            preferred_element_type=f32)
        dist = (rc[:, 0:1] + rj[0:1, :] - 2.0 * dot) / c
        kf = kf + att_ref[0, l] * jnp.exp(-g * dist)
    kf_ref[...] = kf

    a = jnp.abs(kf)

    @pl.when(i == 0)
    def _():
        diagc_ref[...] = jnp.zeros_like(diagc_ref)

    @pl.when(j == i)
    def _():
        eye = (lax.broadcasted_iota(jnp.int32, (bm, bm), 0) ==
               lax.broadcasted_iota(jnp.int32, (bm, bm), 1))
        dcol = jnp.sum(jnp.where(eye, a, 0.0), axis=1, keepdims=True)
        diagc_ref[...] += jnp.broadcast_to(dcol, diagc_ref.shape)

    BIG = 3.4e38
    posmin = jnp.min(jnp.where(a > 0, a, BIG))

    @pl.when(i == 0)
    def _():
        minp_ref[...] = jnp.full_like(minp_ref, BIG)

    minp_ref[...] = jnp.minimum(minp_ref[...], posmin)


def tc_gip_half(yn1, yn2, yn3, rn2c, rn2t, sim, att, half):
    HS = M_SIZE
    bm = 512
    ng = HS // bm
    lo = half * HS

    def sl(x):
        return x[lo:lo + HS]

    def slt(x):
        return x[:, lo:lo + HS]

    y_specs = []
    y_args = []
    for yn, F in ((yn1, F1), (yn2, F2), (yn3, F3)):
        y_args += [sl(yn), sl(yn)]
        y_specs += [
            pl.BlockSpec((bm, F), lambda j, i: (i, 0)),
            pl.BlockSpec((bm, F), lambda j, i: (j, 0)),
        ]
    r_specs_c = [pl.BlockSpec((bm, 8), lambda j, i: (i, 0))] * 3
    r_specs_f = [pl.BlockSpec((16, HS), lambda j, i: (0, 0))] * 3
    r_specs_j = [pl.BlockSpec((16, bm), lambda j, i: (0, j))] * 3
    rc_args = [sl(rn2c[0]), sl(rn2c[1]), sl(rn2c[2])]
    rt_args = [slt(rn2t[0]), slt(rn2t[1]), slt(rn2t[2])]

    return pl.pallas_call(
        functools.partial(_gip_body, HS, bm),
        grid=(ng, ng),
        in_specs=([pl.BlockSpec(memory_space=pltpu.SMEM)] + y_specs +
                  r_specs_c + r_specs_f + r_specs_j +
                  [pl.BlockSpec((bm, bm), lambda j, i: (i, j))]),
        out_specs=[
            pl.BlockSpec((bm, bm), lambda j, i: (i, j)),
            pl.BlockSpec((bm, 8), lambda j, i: (j, 0)),
            pl.BlockSpec((16, bm), lambda j, i: (0, j)),
        ],
        out_shape=[
            jax.ShapeDtypeStruct((HS, HS), f32),
            jax.ShapeDtypeStruct((HS, 8), f32),
            jax.ShapeDtypeStruct((16, HS), f32),
        ],
    )(att, *y_args, *rc_args, *rt_args, *rt_args, sim)


def _minred_body(x_ref, o_ref):
    o_ref[0, 0] = jnp.min(x_ref[...])


def tc_minreduce(minp):
    return pl.pallas_call(
        _minred_body,
        in_specs=[pl.BlockSpec(minp.shape, lambda: (0, 0))],
        out_specs=pl.BlockSpec(memory_space=pltpu.SMEM),
        out_shape=jax.ShapeDtypeStruct((1, 1), f32),
    )(minp)


def _final_body(nk, mpm_ref, mpd_ref, km_ref, a1_ref, ddm_ref,
                kd_ref, a2_ref, ddd_ref, o_ref):
    k = pl.program_id(2)

    @pl.when(k == 0)
    def _():
        o_ref[...] = jnp.zeros_like(o_ref)

    mpm = mpm_ref[0, 0]
    mpd = mpd_ref[0, 0]

    am = jnp.abs(km_ref[...])
    a2m = jnp.where(am == 0.0, mpm, am)
    ddm = ddm_ref[:, 0:1]
    ddm = jnp.where(ddm == 0.0, mpm, ddm)
    a1s = a1_ref[...] / ddm

    ad = jnp.abs(kd_ref[...])
    a2d = jnp.where(ad == 0.0, mpd, ad)
    ddd = ddd_ref[:, 0:1]
    ddd = jnp.where(ddd == 0.0, mpd, ddd)
    a2s = a2_ref[...] / ddd

    t1 = _dot3(a2m, a1s, ((1,), (0,)))
    t2 = _dot3(a2s, a2d, ((0,), (1,)))
    o_ref[...] += 0.5 * (t1 + t2)


def tc_final(km, ddm, mpm, kd, ddd, mpd, alpha1, alpha2):
    bm = 1024
    ng = M_SIZE // bm
    return pl.pallas_call(
        functools.partial(_final_body, ng),
        grid=(ng, ng, ng),
        in_specs=[
            pl.BlockSpec(memory_space=pltpu.SMEM),
            pl.BlockSpec(memory_space=pltpu.SMEM),
            pl.BlockSpec((bm, bm), lambda i, j, k: (i, k)),
            pl.BlockSpec((bm, bm), lambda i, j, k: (k, j)),
            pl.BlockSpec((bm, 8), lambda i, j, k: (k, 0)),
            pl.BlockSpec((bm, bm), lambda i, j, k: (j, k)),
            pl.BlockSpec((bm, bm), lambda i, j, k: (k, i)),
            pl.BlockSpec((bm, 8), lambda i, j, k: (k, 0)),
        ],
        out_specs=pl.BlockSpec((bm, bm), lambda i, j, k: (i, j)),
        out_shape=jax.ShapeDtypeStruct((M_SIZE, M_SIZE), f32),
    )(mpm, mpd, km, alpha1, ddm, kd, alpha2, ddd)


def _gat_layer(x, w, a_src, a_dst, b, src, dst):
    haug, s2 = tc_linear(x, w, a_src, a_dst)
    acc = sc_edge(haug, s2, src, dst)
    return tc_finalize(acc, b)


@jax.jit
def kernel(feature, edge_index, mirna_sim, disease_sim,
           W1, a1_src, a1_dst, b1,
           W2, a2_src, a2_dst, b2,
           W3, a3_src, a3_dst, b3,
           att_m, att_d, alpha1, alpha2):
    loop = jnp.arange(N, dtype=edge_index.dtype)
    src = jnp.concatenate([edge_index[0], loop])
    dst = jnp.concatenate([edge_index[1], loop])

    H1, yn1, rc1, rt1 = _gat_layer(feature, W1, a1_src, a1_dst, b1, src, dst)
    H2, yn2, rc2, rt2 = _gat_layer(H1, W2, a2_src, a2_dst, b2, src, dst)
    H3, yn3, rc3, rt3 = _gat_layer(H2, W3, a3_src, a3_dst, b3, src, dst)

    km, dcm, mpm_p = tc_gip_half(yn1, yn2, yn3, (rc1, rc2, rc3),
                                 (rt1, rt2, rt3), mirna_sim, att_m, 0)
    kd, dcd, mpd_p = tc_gip_half(yn1, yn2, yn3, (rc1, rc2, rc3),
                                 (rt1, rt2, rt3), disease_sim, att_d, 1)
    mpm = tc_minreduce(mpm_p)
    mpd = tc_minreduce(mpd_p)
    return tc_final(km, dcm, mpm, kd, dcd, mpd, alpha1, alpha2)
